# baseline (speedup 1.0000x reference)
import functools

import jax
import jax.numpy as jnp
from jax import lax
from jax.experimental import pallas as pl
from jax.experimental.pallas import tpu as pltpu
from jax.experimental.pallas import tpu_sc as plsc

N = 10000
D = 128
H = 128
E = 320000
EQ = 320000
DE = 16

NP = 10240
NC = 2
NS = 16
NW = NC * NS
K = 80
ROWS_PER_S = NP // NS
NCHUNK = E // NW // K

_mesh = plsc.VectorSubcoreMesh(core_axis_name="c", subcore_axis_name="s")
_sc_params = pltpu.CompilerParams(use_tc_tiling_on_sc=False)


def _make_seg_sum(want_deg):
    out_type = [jax.ShapeDtypeStruct((NC, NP, H), jnp.float32)]
    scratch = [
        pltpu.VMEM((K,), jnp.int32), pltpu.VMEM((K,), jnp.int32),
        pltpu.VMEM((K,), jnp.int32), pltpu.VMEM((K,), jnp.int32),
        pltpu.VMEM((K,), jnp.int32), pltpu.VMEM((K,), jnp.int32),
        pltpu.VMEM((K, H), jnp.float32), pltpu.VMEM((K, H), jnp.float32),
        pltpu.VMEM_SHARED((NP, H), jnp.float32),
        pltpu.SemaphoreType.DMA, pltpu.SemaphoreType.DMA,
        pltpu.SemaphoreType.DMA, pltpu.SemaphoreType.DMA,
        pltpu.SemaphoreType.DMA, pltpu.SemaphoreType.DMA,
        pltpu.SemaphoreType.DMA,
    ]
    if want_deg:
        out_type.append(jax.ShapeDtypeStruct((NC, NP, 16), jnp.float32))
        scratch += [
            pltpu.VMEM((K, 16), jnp.float32),
            pltpu.VMEM_SHARED((NP, 16), jnp.float32),
            pltpu.SemaphoreType.DMA,
        ]

    def body(x_hbm, src_hbm, dst_hbm, zrows_hbm, zdeg_hbm, onesrow_hbm,
             *refs):
        if want_deg:
            (agg_out, deg_out,
             si0, di0, si1, di1, si2, di2, rows0, rows1, agg_sh,
             g0, g1, s0, s1, i0, i1, i2, ones_v, deg_sh, dsem) = refs
        else:
            (agg_out,
             si0, di0, si1, di1, si2, di2, rows0, rows1, agg_sh,
             g0, g1, s0, s1, i0, i1, i2) = refs
        sidx = [si0, si1, si2]
        didx = [di0, di1, di2]
        rows = [rows0, rows1]
        gsem = [g0, g1]
        ssem = [s0, s1]
        isem = [i0, i1, i2]
        c = lax.axis_index("c")
        s = lax.axis_index("s")
        w = s * NC + c
        base0 = w * (E // NW)
        n = E // NW // K

        sl = pl.ds(s * ROWS_PER_S, ROWS_PER_S)
        pltpu.sync_copy(zrows_hbm.at[sl], agg_sh.at[sl])
        if want_deg:
            pltpu.sync_copy(zdeg_hbm.at[sl], deg_sh.at[sl])
            pltpu.sync_copy(onesrow_hbm, ones_v)
        plsc.subcore_barrier()

        def fire_idx(i, p, base=None):
            b = base0 + i * K if base is None else base
            pltpu.async_copy(src_hbm.at[pl.ds(b, K)], sidx[p], isem[p])
            pltpu.async_copy(dst_hbm.at[pl.ds(b, K)], didx[p], isem[p])

        def wait_idx(p):
            pltpu.make_async_copy(src_hbm.at[pl.ds(base0, K)], sidx[p],
                                  isem[p]).wait()
            pltpu.make_async_copy(dst_hbm.at[pl.ds(base0, K)], didx[p],
                                  isem[p]).wait()

        def fire_g(p, b):
            pltpu.async_copy(x_hbm.at[sidx[p]], rows[b], gsem[b])

        def wait_g(p, b):
            pltpu.make_async_copy(x_hbm.at[sidx[p]], rows[b], gsem[b]).wait()

        def fire_s(p, b):
            pltpu.async_copy(rows[b], agg_sh.at[didx[p]], ssem[b], add=True)

        def wait_s(p, b):
            pltpu.make_async_copy(rows[b], agg_sh.at[didx[p]], ssem[b]).wait()

        def fire_deg(p):
            if want_deg:
                pltpu.async_copy(ones_v, deg_sh.at[didx[p]], dsem, add=True)

        def wait_deg(p):
            if want_deg:
                pltpu.make_async_copy(ones_v, deg_sh.at[didx[p]], dsem).wait()

        def step(ci, pk, fire_next_g, fire_next_idx, drain_prev):
            p, b = pk % 3, pk % 2
            wait_g(p, b)
            fire_s(p, b)
            fire_deg(p)
            if drain_prev:
                wait_s((pk - 1) % 3, (pk - 1) % 2)
                wait_deg((pk - 1) % 3)
            if fire_next_g:
                wait_idx((pk + 1) % 3)
                fire_g((pk + 1) % 3, (pk + 1) % 2)
            if fire_next_idx:
                fire_idx(ci + 2, (pk + 2) % 3)

        fire_idx(0, 0)
        fire_idx(1, 1)
        wait_idx(0)
        fire_g(0, 0)
        step(0, 0, True, True, False)
        step(1, 1, True, True, True)

        def six(j, carry):
            ci0 = 6 * j + 2
            for k in range(6):
                step(ci0 + k, 2 + k, True, True, True)
            return carry

        lax.fori_loop(0, (n - 5) // 6, six, 0)

        step(n - 3, n - 3, True, True, True)
        step(n - 2, n - 2, True, False, True)
        step(n - 1, n - 1, False, False, True)
        wait_s((n - 1) % 3, (n - 1) % 2)
        wait_deg((n - 1) % 3)

        plsc.subcore_barrier()
        pltpu.sync_copy(agg_sh.at[sl], agg_out.at[c, sl])
        if want_deg:
            pltpu.sync_copy(deg_sh.at[sl], deg_out.at[c, sl])

    return pl.kernel(body, out_type=out_type, mesh=_mesh,
                     scratch_types=scratch, compiler_params=_sc_params)


_sc_seg_sum_deg = _make_seg_sum(True)
_sc_seg_sum = _make_seg_sum(False)


@functools.partial(
    pl.kernel,
    out_type=jax.ShapeDtypeStruct((EQ, 64), jnp.bfloat16),
    mesh=_mesh,
    scratch_types=[
        pltpu.VMEM((K,), jnp.int32), pltpu.VMEM((K,), jnp.int32),
        pltpu.VMEM((K,), jnp.int32), pltpu.VMEM((K,), jnp.int32),
        pltpu.VMEM((K,), jnp.int32), pltpu.VMEM((K,), jnp.int32),
        pltpu.VMEM((K, 64), jnp.bfloat16), pltpu.VMEM((K, 64), jnp.bfloat16),
        pltpu.VMEM((K, 64), jnp.bfloat16), pltpu.VMEM((K, 64), jnp.bfloat16),
        pltpu.SemaphoreType.DMA, pltpu.SemaphoreType.DMA,
        pltpu.SemaphoreType.DMA, pltpu.SemaphoreType.DMA,
        pltpu.SemaphoreType.DMA, pltpu.SemaphoreType.DMA,
        pltpu.SemaphoreType.DMA, pltpu.SemaphoreType.DMA,
        pltpu.SemaphoreType.DMA,
    ],
    compiler_params=_sc_params,
)
def _sc_query_gather(a_hbm, b_hbm, qsrc_hbm, qdst_hbm,
                     q_out,
                     si0, di0, si1, di1, si2, di2, a0, b0, a1, b1,
                     ga0, gb0, ga1, gb1, o0, o1, is0, is1, is2):
    sidx = [si0, si1, si2]
    didx = [di0, di1, di2]
    av = [a0, a1]
    bv = [b0, b1]
    gas = [ga0, ga1]
    gbs = [gb0, gb1]
    osem = [o0, o1]
    isem = [is0, is1, is2]
    c = lax.axis_index("c")
    s = lax.axis_index("s")
    w = s * NC + c
    base0 = w * (EQ // NW)
    n = EQ // NW // K

    def fire_idx(i, p):
        b = base0 + i * K
        pltpu.async_copy(qsrc_hbm.at[pl.ds(b, K)], sidx[p], isem[p])
        pltpu.async_copy(qdst_hbm.at[pl.ds(b, K)], didx[p], isem[p])

    def wait_idx(p):
        pltpu.make_async_copy(qsrc_hbm.at[pl.ds(base0, K)], sidx[p],
                              isem[p]).wait()
        pltpu.make_async_copy(qdst_hbm.at[pl.ds(base0, K)], didx[p],
                              isem[p]).wait()

    def fire_g(p, b):
        pltpu.async_copy(a_hbm.at[sidx[p]], av[b], gas[b])
        pltpu.async_copy(b_hbm.at[didx[p]], bv[b], gbs[b])

    def wait_g(p, b):
        pltpu.make_async_copy(a_hbm.at[sidx[p]], av[b], gas[b]).wait()
        pltpu.make_async_copy(b_hbm.at[didx[p]], bv[b], gbs[b]).wait()

    def add_rows(b):
        a_v, b_v = av[b], bv[b]

        def row(r, carry):
            for l in range(2):
                slc = pl.ds(l * 32, 32)
                a_v[r, slc] = a_v[r, slc] + b_v[r, slc]
            return carry
        lax.fori_loop(0, K, row, 0)

    def fire_out(ci, b):
        pltpu.async_copy(av[b], q_out.at[pl.ds(base0 + ci * K, K)], osem[b])

    def wait_out(b):
        pltpu.make_async_copy(av[b], q_out.at[pl.ds(base0, K)],
                              osem[b]).wait()

    def step(ci, pk, fire_next_g, fire_next_idx, drain_prev):
        p, b = pk % 3, pk % 2
        wait_g(p, b)
        if drain_prev:
            wait_out((pk - 1) % 2)
        if fire_next_g:
            wait_idx((pk + 1) % 3)
            fire_g((pk + 1) % 3, (pk + 1) % 2)
        if fire_next_idx:
            fire_idx(ci + 2, (pk + 2) % 3)
        add_rows(b)
        fire_out(ci, b)

    fire_idx(0, 0)
    fire_idx(1, 1)
    wait_idx(0)
    fire_g(0, 0)
    step(0, 0, True, True, False)
    step(1, 1, True, True, True)

    def six(j, carry):
        ci0 = 6 * j + 2
        for k in range(6):
            step(ci0 + k, 2 + k, True, True, True)
        return carry

    lax.fori_loop(0, (n - 5) // 6, six, 0)

    step(n - 3, n - 3, True, True, True)
    step(n - 2, n - 2, True, False, True)
    step(n - 1, n - 1, False, False, True)
    wait_out((n - 1) % 2)


BN = 1024


def _tc1_body(aggp_ref, degp_ref, x_ref, wl_ref, bl_ref, wr_ref, h1_ref):
    agg = aggp_ref[0] + aggp_ref[1]
    deg = degp_ref[0, :, 0] + degp_ref[1, :, 0]
    rdeg = 1.0 / jnp.maximum(deg, 1.0)
    mean = agg * rdeg[:, None]
    out = (jnp.dot(mean, wl_ref[...], preferred_element_type=jnp.float32)
           + bl_ref[...]
           + jnp.dot(x_ref[...], wr_ref[...], preferred_element_type=jnp.float32))
    h1_ref[...] = jnp.maximum(out, 0.0)


def _tc1(aggp, degp, x, wl, bl, wr):
    return pl.pallas_call(
        _tc1_body,
        grid=(NP // BN,),
        in_specs=[
            pl.BlockSpec((NC, BN, H), lambda i: (0, i, 0)),
            pl.BlockSpec((NC, BN, 16), lambda i: (0, i, 0)),
            pl.BlockSpec((BN, H), lambda i: (i, 0)),
            pl.BlockSpec((H, H), lambda i: (0, 0)),
            pl.BlockSpec((1, H), lambda i: (0, 0)),
            pl.BlockSpec((H, H), lambda i: (0, 0)),
        ],
        out_specs=pl.BlockSpec((BN, H), lambda i: (i, 0)),
        out_shape=jax.ShapeDtypeStruct((NP, H), jnp.float32),
    )(aggp, degp, x, wl, bl, wr)


def _tc2_body(aggp_ref, degp_ref, h1_ref, wl_ref, bl_ref, wr_ref, wsd_ref,
              a_ref, b_ref):
    agg = aggp_ref[0] + aggp_ref[1]
    deg = degp_ref[0, :, 0] + degp_ref[1, :, 0]
    rdeg = 1.0 / jnp.maximum(deg, 1.0)
    mean = agg * rdeg[:, None]
    h2 = (jnp.dot(mean, wl_ref[...], preferred_element_type=jnp.float32)
          + bl_ref[...]
          + jnp.dot(h1_ref[...], wr_ref[...], preferred_element_type=jnp.float32))
    ab = jnp.dot(h2, wsd_ref[...], preferred_element_type=jnp.float32)
    a_ref[...] = ab[:, :64].astype(jnp.bfloat16)
    b_ref[...] = ab[:, 64:].astype(jnp.bfloat16)


def _tc2(aggp, degp, h1, wl, bl, wr, wsd):
    return pl.pallas_call(
        _tc2_body,
        grid=(NP // BN,),
        in_specs=[
            pl.BlockSpec((NC, BN, H), lambda i: (0, i, 0)),
            pl.BlockSpec((NC, BN, 16), lambda i: (0, i, 0)),
            pl.BlockSpec((BN, H), lambda i: (i, 0)),
            pl.BlockSpec((H, H), lambda i: (0, 0)),
            pl.BlockSpec((1, H), lambda i: (0, 0)),
            pl.BlockSpec((H, H), lambda i: (0, 0)),
            pl.BlockSpec((H, H), lambda i: (0, 0)),
        ],
        out_specs=[
            pl.BlockSpec((BN, 64), lambda i: (i, 0)),
            pl.BlockSpec((BN, 64), lambda i: (i, 0)),
        ],
        out_shape=[
            jax.ShapeDtypeStruct((NP, 64), jnp.bfloat16),
            jax.ShapeDtypeStruct((NP, 64), jnp.bfloat16),
        ],
    )(aggp, degp, h1, wl, bl, wr, wsd)


BE = 2000
GE = EQ // BE
BR = 8


def _tc_mlp_body(q_ref, attr_ref, wa_ref, bc1_ref, wc2_ref, bc2_ref,
                 wc3_ref, bc3_ref, out_ref):
    q = q_ref[...].astype(jnp.float32).reshape(BR * BE, 64)
    attr = attr_ref[...].reshape(BR * BE, DE)
    z1 = q + jnp.dot(attr, wa_ref[...], preferred_element_type=jnp.float32) + bc1_ref[...]
    z1 = jnp.maximum(z1, 0.0)
    z2 = jnp.dot(z1, wc2_ref[...], preferred_element_type=jnp.float32) + bc2_ref[...]
    z2 = jnp.maximum(z2, 0.0)
    z3 = jnp.sum(z2 * wc3_ref[...], axis=1) + bc3_ref[0, 0]
    out_ref[...] = z3.reshape(BR, BE)


def _tc_mlp(q3, attr3, wa, bc1, wc2, bc2, wc3, bc3):
    return pl.pallas_call(
        _tc_mlp_body,
        grid=(GE // BR,),
        in_specs=[
            pl.BlockSpec((BR, BE, 64), lambda i: (i, 0, 0)),
            pl.BlockSpec((BR, BE, DE), lambda i: (i, 0, 0)),
            pl.BlockSpec((DE, 64), lambda i: (0, 0)),
            pl.BlockSpec((1, 64), lambda i: (0, 0)),
            pl.BlockSpec((64, 32), lambda i: (0, 0)),
            pl.BlockSpec((1, 32), lambda i: (0, 0)),
            pl.BlockSpec((1, 32), lambda i: (0, 0)),
            pl.BlockSpec((1, 1), lambda i: (0, 0)),
        ],
        out_specs=pl.BlockSpec((BR, BE), lambda i: (i, 0)),
        out_shape=jax.ShapeDtypeStruct((GE, BE), jnp.float32),
    )(q3, attr3, wa, bc1, wc2, bc2, wc3, bc3)


def kernel(x, message_edge_index, query_edge_index, query_edge_attr,
           W1l, b1l, W1r, W2l, b2l, W2r,
           Wc1, bc1, Wc2, bc2, Wc3, bc3):
    x_p = jnp.pad(x, ((0, NP - N), (0, 0)))
    src = message_edge_index[0]
    dst = message_edge_index[1]
    qsrc = query_edge_index[0]
    qdst = query_edge_index[1]
    zrows = jnp.zeros((NP, H), jnp.float32)
    zdeg = jnp.zeros((NP, 16), jnp.float32)
    onesrow = jnp.zeros((K, 16), jnp.float32).at[:, 0].set(1.0)

    agg1p, degp = _sc_seg_sum_deg(x_p, src, dst, zrows, zdeg, onesrow)
    h1 = _tc1(agg1p, degp, x_p, W1l, b1l[None, :], W1r)

    agg2p, = _sc_seg_sum(h1, src, dst, zrows, zdeg, onesrow)
    Wsd = jnp.concatenate([Wc1[:H], Wc1[H:2 * H]], axis=1)
    a_tab, b_tab = _tc2(agg2p, degp, h1, W2l, b2l[None, :], W2r, Wsd)

    q = _sc_query_gather(a_tab, b_tab, qsrc, qdst)
    out3 = _tc_mlp(q.reshape(GE // BR, BR, BE, 64).reshape(GE, BE, 64),
                   query_edge_attr.reshape(GE, BE, DE),
                   Wc1[2 * H:], bc1[None, :], Wc2, bc2[None, :],
                   Wc3.reshape(1, 32), bc3.reshape(1, 1))
    return out3.reshape(EQ)


# speedup vs baseline: 1.0327x; 1.0327x over previous
import functools

import jax
import jax.numpy as jnp
from jax import lax
from jax.experimental import pallas as pl
from jax.experimental.pallas import tpu as pltpu
from jax.experimental.pallas import tpu_sc as plsc

N = 10000
D = 128
H = 128
E = 320000
EQ = 320000
DE = 16

NP = 10240
NC = 2
NS = 16
NW = NC * NS
K = 80
ROWS_PER_S = NP // NS
NCHUNK = E // NW // K

_mesh = plsc.VectorSubcoreMesh(core_axis_name="c", subcore_axis_name="s")
_sc_params = pltpu.CompilerParams(use_tc_tiling_on_sc=False)


def _make_seg_sum(want_deg):
    out_type = [jax.ShapeDtypeStruct((NC, NP, H), jnp.float32)]
    scratch = [
        pltpu.VMEM((K,), jnp.int32), pltpu.VMEM((K,), jnp.int32),
        pltpu.VMEM((K,), jnp.int32), pltpu.VMEM((K,), jnp.int32),
        pltpu.VMEM((K,), jnp.int32), pltpu.VMEM((K,), jnp.int32),
        pltpu.VMEM((K, H), jnp.float32), pltpu.VMEM((K, H), jnp.float32),
        pltpu.VMEM_SHARED((NP, H), jnp.float32),
        pltpu.SemaphoreType.DMA, pltpu.SemaphoreType.DMA,
        pltpu.SemaphoreType.DMA, pltpu.SemaphoreType.DMA,
        pltpu.SemaphoreType.DMA, pltpu.SemaphoreType.DMA,
        pltpu.SemaphoreType.DMA,
    ]
    if want_deg:
        out_type.append(jax.ShapeDtypeStruct((NC, NP, 16), jnp.float32))
        scratch += [
            pltpu.VMEM((K, 16), jnp.float32),
            pltpu.VMEM_SHARED((NP, 16), jnp.float32),
            pltpu.SemaphoreType.DMA,
        ]

    def body(x_hbm, src_hbm, dst_hbm, zrows_hbm, zdeg_hbm, onesrow_hbm,
             *refs):
        if want_deg:
            (agg_out, deg_out,
             si0, di0, si1, di1, si2, di2, rows0, rows1, agg_sh,
             g0, g1, s0, s1, i0, i1, i2, ones_v, deg_sh, dsem) = refs
        else:
            (agg_out,
             si0, di0, si1, di1, si2, di2, rows0, rows1, agg_sh,
             g0, g1, s0, s1, i0, i1, i2) = refs
        sidx = [si0, si1, si2]
        didx = [di0, di1, di2]
        rows = [rows0, rows1]
        gsem = [g0, g1]
        ssem = [s0, s1]
        isem = [i0, i1, i2]
        c = lax.axis_index("c")
        s = lax.axis_index("s")
        w = s * NC + c
        base0 = w * (E // NW)
        n = E // NW // K

        sl = pl.ds(s * ROWS_PER_S, ROWS_PER_S)
        pltpu.sync_copy(zrows_hbm.at[sl], agg_sh.at[sl])
        if want_deg:
            pltpu.sync_copy(zdeg_hbm.at[sl], deg_sh.at[sl])
            pltpu.sync_copy(onesrow_hbm, ones_v)
        plsc.subcore_barrier()

        def fire_idx(i, p, base=None):
            b = base0 + i * K if base is None else base
            pltpu.async_copy(src_hbm.at[pl.ds(b, K)], sidx[p], isem[p])
            pltpu.async_copy(dst_hbm.at[pl.ds(b, K)], didx[p], isem[p])

        def wait_idx(p):
            pltpu.make_async_copy(src_hbm.at[pl.ds(base0, K)], sidx[p],
                                  isem[p]).wait()
            pltpu.make_async_copy(dst_hbm.at[pl.ds(base0, K)], didx[p],
                                  isem[p]).wait()

        def fire_g(p, b):
            pltpu.async_copy(x_hbm.at[sidx[p]], rows[b], gsem[b])

        def wait_g(p, b):
            pltpu.make_async_copy(x_hbm.at[sidx[p]], rows[b], gsem[b]).wait()

        def fire_s(p, b):
            pltpu.async_copy(rows[b], agg_sh.at[didx[p]], ssem[b], add=True)

        def wait_s(p, b):
            pltpu.make_async_copy(rows[b], agg_sh.at[didx[p]], ssem[b]).wait()

        def fire_deg(p):
            if want_deg:
                pltpu.async_copy(ones_v, deg_sh.at[didx[p]], dsem, add=True)

        def wait_deg(p):
            if want_deg:
                pltpu.make_async_copy(ones_v, deg_sh.at[didx[p]], dsem).wait()

        def step(ci, pk, fire_next_g, fire_next_idx, drain_prev):
            p, b = pk % 3, pk % 2
            wait_g(p, b)
            fire_s(p, b)
            fire_deg(p)
            if drain_prev:
                wait_s((pk - 1) % 3, (pk - 1) % 2)
                wait_deg((pk - 1) % 3)
            if fire_next_g:
                wait_idx((pk + 1) % 3)
                fire_g((pk + 1) % 3, (pk + 1) % 2)
            if fire_next_idx:
                fire_idx(ci + 2, (pk + 2) % 3)

        fire_idx(0, 0)
        fire_idx(1, 1)
        wait_idx(0)
        fire_g(0, 0)
        step(0, 0, True, True, False)
        step(1, 1, True, True, True)

        def six(j, carry):
            ci0 = 6 * j + 2
            for k in range(6):
                step(ci0 + k, 2 + k, True, True, True)
            return carry

        lax.fori_loop(0, (n - 5) // 6, six, 0)

        step(n - 3, n - 3, True, True, True)
        step(n - 2, n - 2, True, False, True)
        step(n - 1, n - 1, False, False, True)
        wait_s((n - 1) % 3, (n - 1) % 2)
        wait_deg((n - 1) % 3)

        plsc.subcore_barrier()
        pltpu.sync_copy(agg_sh.at[sl], agg_out.at[c, sl])
        if want_deg:
            pltpu.sync_copy(deg_sh.at[sl], deg_out.at[c, sl])

    return pl.kernel(body, out_type=out_type, mesh=_mesh,
                     scratch_types=scratch, compiler_params=_sc_params)


_sc_seg_sum_deg = _make_seg_sum(True)
_sc_seg_sum = _make_seg_sum(False)


KH = K // 2


@functools.partial(
    pl.kernel,
    out_type=jax.ShapeDtypeStruct((EQ // 2, 128), jnp.float32),
    mesh=_mesh,
    scratch_types=[
        pltpu.VMEM((K,), jnp.int32), pltpu.VMEM((K,), jnp.int32),
        pltpu.VMEM((K,), jnp.int32), pltpu.VMEM((K,), jnp.int32),
        pltpu.VMEM((K,), jnp.int32), pltpu.VMEM((K,), jnp.int32),
        pltpu.VMEM((K, 64), jnp.float32), pltpu.VMEM((K, 64), jnp.float32),
        pltpu.VMEM((K, 64), jnp.float32), pltpu.VMEM((K, 64), jnp.float32),
        pltpu.VMEM((KH, 128), jnp.float32), pltpu.VMEM((KH, 128), jnp.float32),
        pltpu.SemaphoreType.DMA, pltpu.SemaphoreType.DMA,
        pltpu.SemaphoreType.DMA, pltpu.SemaphoreType.DMA,
        pltpu.SemaphoreType.DMA, pltpu.SemaphoreType.DMA,
        pltpu.SemaphoreType.DMA, pltpu.SemaphoreType.DMA,
        pltpu.SemaphoreType.DMA,
    ],
    compiler_params=_sc_params,
)
def _sc_query_gather(a_hbm, b_hbm, qsrc_hbm, qdst_hbm,
                     q_out,
                     si0, di0, si1, di1, si2, di2, a0, b0, a1, b1,
                     o_v0, o_v1,
                     ga0, gb0, ga1, gb1, o0, o1, is0, is1, is2):
    sidx = [si0, si1, si2]
    didx = [di0, di1, di2]
    av = [a0, a1]
    bv = [b0, b1]
    ov = [o_v0, o_v1]
    gas = [ga0, ga1]
    gbs = [gb0, gb1]
    osem = [o0, o1]
    isem = [is0, is1, is2]
    c = lax.axis_index("c")
    s = lax.axis_index("s")
    w = s * NC + c
    base0 = w * (EQ // NW)
    hbase0 = w * (EQ // NW // 2)
    n = EQ // NW // K

    def fire_idx(i, p):
        b = base0 + i * K
        pltpu.async_copy(qsrc_hbm.at[pl.ds(b, K)], sidx[p], isem[p])
        pltpu.async_copy(qdst_hbm.at[pl.ds(b, K)], didx[p], isem[p])

    def wait_idx(p):
        pltpu.make_async_copy(qsrc_hbm.at[pl.ds(base0, K)], sidx[p],
                              isem[p]).wait()
        pltpu.make_async_copy(qdst_hbm.at[pl.ds(base0, K)], didx[p],
                              isem[p]).wait()

    def fire_g(p, b):
        pltpu.async_copy(a_hbm.at[sidx[p]], av[b], gas[b])
        pltpu.async_copy(b_hbm.at[didx[p]], bv[b], gbs[b])

    def wait_g(p, b):
        pltpu.make_async_copy(a_hbm.at[sidx[p]], av[b], gas[b]).wait()
        pltpu.make_async_copy(b_hbm.at[didx[p]], bv[b], gbs[b]).wait()

    def add_rows(b):
        a_v, b_v, o_v = av[b], bv[b], ov[b]

        def row(r2, carry):
            for l in range(8):
                half = l // 4
                slc = pl.ds((l % 4) * 16, 16)
                o_v[r2, pl.ds(l * 16, 16)] = (a_v[2 * r2 + half, slc]
                                              + b_v[2 * r2 + half, slc])
            return carry
        lax.fori_loop(0, KH, row, 0)

    def fire_out(ci, b):
        pltpu.async_copy(ov[b], q_out.at[pl.ds(hbase0 + ci * KH, KH)],
                         osem[b])

    def wait_out(b):
        pltpu.make_async_copy(ov[b], q_out.at[pl.ds(hbase0, KH)],
                              osem[b]).wait()

    def step(ci, pk, fire_next_g, fire_next_idx, drain_prev):
        p, b = pk % 3, pk % 2
        wait_g(p, b)
        if drain_prev:
            wait_out((pk - 1) % 2)
        if fire_next_g:
            wait_idx((pk + 1) % 3)
            fire_g((pk + 1) % 3, (pk + 1) % 2)
        if fire_next_idx:
            fire_idx(ci + 2, (pk + 2) % 3)
        add_rows(b)
        fire_out(ci, b)

    fire_idx(0, 0)
    fire_idx(1, 1)
    wait_idx(0)
    fire_g(0, 0)
    step(0, 0, True, True, False)
    step(1, 1, True, True, True)

    def six(j, carry):
        ci0 = 6 * j + 2
        for k in range(6):
            step(ci0 + k, 2 + k, True, True, True)
        return carry

    lax.fori_loop(0, (n - 5) // 6, six, 0)

    step(n - 3, n - 3, True, True, True)
    step(n - 2, n - 2, True, False, True)
    step(n - 1, n - 1, False, False, True)
    wait_out((n - 1) % 2)


BN = 1024


def _tc1_body(aggp_ref, degp_ref, x_ref, wl_ref, bl_ref, wr_ref, h1_ref):
    agg = aggp_ref[0] + aggp_ref[1]
    deg = degp_ref[0, :, 0] + degp_ref[1, :, 0]
    rdeg = 1.0 / jnp.maximum(deg, 1.0)
    mean = agg * rdeg[:, None]
    out = (jnp.dot(mean, wl_ref[...], preferred_element_type=jnp.float32)
           + bl_ref[...]
           + jnp.dot(x_ref[...], wr_ref[...], preferred_element_type=jnp.float32))
    h1_ref[...] = jnp.maximum(out, 0.0)


def _tc1(aggp, degp, x, wl, bl, wr):
    return pl.pallas_call(
        _tc1_body,
        grid=(NP // BN,),
        in_specs=[
            pl.BlockSpec((NC, BN, H), lambda i: (0, i, 0)),
            pl.BlockSpec((NC, BN, 16), lambda i: (0, i, 0)),
            pl.BlockSpec((BN, H), lambda i: (i, 0)),
            pl.BlockSpec((H, H), lambda i: (0, 0)),
            pl.BlockSpec((1, H), lambda i: (0, 0)),
            pl.BlockSpec((H, H), lambda i: (0, 0)),
        ],
        out_specs=pl.BlockSpec((BN, H), lambda i: (i, 0)),
        out_shape=jax.ShapeDtypeStruct((NP, H), jnp.float32),
    )(aggp, degp, x, wl, bl, wr)


def _tc2_body(aggp_ref, degp_ref, h1_ref, wl_ref, bl_ref, wr_ref, wsd_ref,
              a_ref, b_ref):
    agg = aggp_ref[0] + aggp_ref[1]
    deg = degp_ref[0, :, 0] + degp_ref[1, :, 0]
    rdeg = 1.0 / jnp.maximum(deg, 1.0)
    mean = agg * rdeg[:, None]
    h2 = (jnp.dot(mean, wl_ref[...], preferred_element_type=jnp.float32)
          + bl_ref[...]
          + jnp.dot(h1_ref[...], wr_ref[...], preferred_element_type=jnp.float32))
    ab = jnp.dot(h2, wsd_ref[...], preferred_element_type=jnp.float32)
    a_ref[...] = ab[:, :64]
    b_ref[...] = ab[:, 64:]


def _tc2(aggp, degp, h1, wl, bl, wr, wsd):
    return pl.pallas_call(
        _tc2_body,
        grid=(NP // BN,),
        in_specs=[
            pl.BlockSpec((NC, BN, H), lambda i: (0, i, 0)),
            pl.BlockSpec((NC, BN, 16), lambda i: (0, i, 0)),
            pl.BlockSpec((BN, H), lambda i: (i, 0)),
            pl.BlockSpec((H, H), lambda i: (0, 0)),
            pl.BlockSpec((1, H), lambda i: (0, 0)),
            pl.BlockSpec((H, H), lambda i: (0, 0)),
            pl.BlockSpec((H, H), lambda i: (0, 0)),
        ],
        out_specs=[
            pl.BlockSpec((BN, 64), lambda i: (i, 0)),
            pl.BlockSpec((BN, 64), lambda i: (i, 0)),
        ],
        out_shape=[
            jax.ShapeDtypeStruct((NP, 64), jnp.float32),
            jax.ShapeDtypeStruct((NP, 64), jnp.float32),
        ],
    )(aggp, degp, h1, wl, bl, wr, wsd)


EH = EQ // 2
BQ = 16000
GQ = EH // BQ
BE = 2000
BR = 8


def _tc_mlp_body(q_ref, attr_ref, w2a_ref, bc1_ref, w2c2_ref, bc2_ref,
                 wc3_ref, bc3_ref, oute_ref, outo_ref):
    q = q_ref[...]
    attr = attr_ref[...]
    z1 = q + jnp.dot(attr, w2a_ref[...], preferred_element_type=jnp.float32) + bc1_ref[...]
    z1 = jnp.maximum(z1, 0.0)
    z2 = jnp.dot(z1, w2c2_ref[...], preferred_element_type=jnp.float32) + bc2_ref[...]
    z2 = jnp.maximum(z2, 0.0)
    ze = jnp.sum(z2[:, :32] * wc3_ref[...], axis=1) + bc3_ref[0, 0]
    zo = jnp.sum(z2[:, 32:] * wc3_ref[...], axis=1) + bc3_ref[0, 0]
    oute_ref[...] = ze.reshape(BR, BE)
    outo_ref[...] = zo.reshape(BR, BE)


def _tc_mlp(q2, attr2, w2a, bc1_2, w2c2, bc2_2, wc3, bc3):
    return pl.pallas_call(
        _tc_mlp_body,
        grid=(GQ,),
        in_specs=[
            pl.BlockSpec((BQ, 128), lambda i: (i, 0)),
            pl.BlockSpec((BQ, 2 * DE), lambda i: (i, 0)),
            pl.BlockSpec((2 * DE, 128), lambda i: (0, 0)),
            pl.BlockSpec((1, 128), lambda i: (0, 0)),
            pl.BlockSpec((128, 64), lambda i: (0, 0)),
            pl.BlockSpec((1, 64), lambda i: (0, 0)),
            pl.BlockSpec((1, 32), lambda i: (0, 0)),
            pl.BlockSpec((1, 1), lambda i: (0, 0)),
        ],
        out_specs=[
            pl.BlockSpec((BR, BE), lambda i: (i, 0)),
            pl.BlockSpec((BR, BE), lambda i: (i, 0)),
        ],
        out_shape=[
            jax.ShapeDtypeStruct((GQ * BR, BE), jnp.float32),
            jax.ShapeDtypeStruct((GQ * BR, BE), jnp.float32),
        ],
    )(q2, attr2, w2a, bc1_2, w2c2, bc2_2, wc3, bc3)


def kernel(x, message_edge_index, query_edge_index, query_edge_attr,
           W1l, b1l, W1r, W2l, b2l, W2r,
           Wc1, bc1, Wc2, bc2, Wc3, bc3):
    x_p = jnp.pad(x, ((0, NP - N), (0, 0)))
    src = message_edge_index[0]
    dst = message_edge_index[1]
    qsrc = query_edge_index[0]
    qdst = query_edge_index[1]
    zrows = jnp.zeros((NP, H), jnp.float32)
    zdeg = jnp.zeros((NP, 16), jnp.float32)
    onesrow = jnp.zeros((K, 16), jnp.float32).at[:, 0].set(1.0)

    agg1p, degp = _sc_seg_sum_deg(x_p, src, dst, zrows, zdeg, onesrow)
    h1 = _tc1(agg1p, degp, x_p, W1l, b1l[None, :], W1r)

    agg2p, = _sc_seg_sum(h1, src, dst, zrows, zdeg, onesrow)
    Wsd = jnp.concatenate([Wc1[:H], Wc1[H:2 * H]], axis=1)
    a_tab, b_tab = _tc2(agg2p, degp, h1, W2l, b2l[None, :], W2r, Wsd)

    q2 = _sc_query_gather(a_tab, b_tab, qsrc, qdst)
    Wa = Wc1[2 * H:]
    w2a = jnp.zeros((2 * DE, 128), jnp.float32)
    w2a = w2a.at[:DE, :64].set(Wa).at[DE:, 64:].set(Wa)
    w2c2 = jnp.zeros((128, 64), jnp.float32)
    w2c2 = w2c2.at[:64, :32].set(Wc2).at[64:, 32:].set(Wc2)
    bc1_2 = jnp.concatenate([bc1, bc1])[None, :]
    bc2_2 = jnp.concatenate([bc2, bc2])[None, :]
    oute, outo = _tc_mlp(q2, query_edge_attr.reshape(EH, 2 * DE),
                         w2a, bc1_2, w2c2, bc2_2,
                         Wc3.reshape(1, 32), bc3.reshape(1, 1))
    out = jnp.stack([oute.reshape(EH), outo.reshape(EH)], axis=-1)
    return out.reshape(EQ)


# speedup vs baseline: 1.0396x; 1.0066x over previous
import functools

import jax
import jax.numpy as jnp
from jax import lax
from jax.experimental import pallas as pl
from jax.experimental.pallas import tpu as pltpu
from jax.experimental.pallas import tpu_sc as plsc

N = 10000
D = 128
H = 128
E = 320000
EQ = 320000
DE = 16

NP = 10000
NC = 2
NS = 16
NW = NC * NS
K = 80
ROWS_PER_S = NP // NS
NCHUNK = E // NW // K

_mesh = plsc.VectorSubcoreMesh(core_axis_name="c", subcore_axis_name="s")
_sc_params = pltpu.CompilerParams(use_tc_tiling_on_sc=False)


def _make_seg_sum(want_deg):
    out_type = [jax.ShapeDtypeStruct((NC, NP, H), jnp.float32)]
    scratch = [
        pltpu.VMEM((K,), jnp.int32), pltpu.VMEM((K,), jnp.int32),
        pltpu.VMEM((K,), jnp.int32), pltpu.VMEM((K,), jnp.int32),
        pltpu.VMEM((K,), jnp.int32), pltpu.VMEM((K,), jnp.int32),
        pltpu.VMEM((K, H), jnp.float32), pltpu.VMEM((K, H), jnp.float32),
        pltpu.VMEM_SHARED((NP, H), jnp.float32),
        pltpu.SemaphoreType.DMA, pltpu.SemaphoreType.DMA,
        pltpu.SemaphoreType.DMA, pltpu.SemaphoreType.DMA,
        pltpu.SemaphoreType.DMA, pltpu.SemaphoreType.DMA,
        pltpu.SemaphoreType.DMA,
    ]
    if want_deg:
        out_type.append(jax.ShapeDtypeStruct((NC, NP, 16), jnp.float32))
        scratch += [
            pltpu.VMEM((K, 16), jnp.float32),
            pltpu.VMEM_SHARED((NP, 16), jnp.float32),
            pltpu.SemaphoreType.DMA,
        ]

    def body(x_hbm, src_hbm, dst_hbm, zrows_hbm, zdeg_hbm, onesrow_hbm,
             *refs):
        if want_deg:
            (agg_out, deg_out,
             si0, di0, si1, di1, si2, di2, rows0, rows1, agg_sh,
             g0, g1, s0, s1, i0, i1, i2, ones_v, deg_sh, dsem) = refs
        else:
            (agg_out,
             si0, di0, si1, di1, si2, di2, rows0, rows1, agg_sh,
             g0, g1, s0, s1, i0, i1, i2) = refs
        sidx = [si0, si1, si2]
        didx = [di0, di1, di2]
        rows = [rows0, rows1]
        gsem = [g0, g1]
        ssem = [s0, s1]
        isem = [i0, i1, i2]
        c = lax.axis_index("c")
        s = lax.axis_index("s")
        w = s * NC + c
        base0 = w * (E // NW)
        n = E // NW // K

        sl = pl.ds(s * ROWS_PER_S, ROWS_PER_S)
        pltpu.sync_copy(zrows_hbm.at[sl], agg_sh.at[sl])
        if want_deg:
            pltpu.sync_copy(zdeg_hbm.at[sl], deg_sh.at[sl])
            pltpu.sync_copy(onesrow_hbm, ones_v)
        plsc.subcore_barrier()

        def fire_idx(i, p, base=None):
            b = base0 + i * K if base is None else base
            pltpu.async_copy(src_hbm.at[pl.ds(b, K)], sidx[p], isem[p])
            pltpu.async_copy(dst_hbm.at[pl.ds(b, K)], didx[p], isem[p])

        def wait_idx(p):
            pltpu.make_async_copy(src_hbm.at[pl.ds(base0, K)], sidx[p],
                                  isem[p]).wait()
            pltpu.make_async_copy(dst_hbm.at[pl.ds(base0, K)], didx[p],
                                  isem[p]).wait()

        def fire_g(p, b):
            pltpu.async_copy(x_hbm.at[sidx[p]], rows[b], gsem[b])

        def wait_g(p, b):
            pltpu.make_async_copy(x_hbm.at[sidx[p]], rows[b], gsem[b]).wait()

        def fire_s(p, b):
            pltpu.async_copy(rows[b], agg_sh.at[didx[p]], ssem[b], add=True)

        def wait_s(p, b):
            pltpu.make_async_copy(rows[b], agg_sh.at[didx[p]], ssem[b]).wait()

        def fire_deg(p):
            if want_deg:
                pltpu.async_copy(ones_v, deg_sh.at[didx[p]], dsem, add=True)

        def wait_deg(p):
            if want_deg:
                pltpu.make_async_copy(ones_v, deg_sh.at[didx[p]], dsem).wait()

        def step(ci, pk, fire_next_g, fire_next_idx, drain_prev):
            p, b = pk % 3, pk % 2
            wait_g(p, b)
            fire_s(p, b)
            fire_deg(p)
            if drain_prev:
                wait_s((pk - 1) % 3, (pk - 1) % 2)
                wait_deg((pk - 1) % 3)
            if fire_next_g:
                wait_idx((pk + 1) % 3)
                fire_g((pk + 1) % 3, (pk + 1) % 2)
            if fire_next_idx:
                fire_idx(ci + 2, (pk + 2) % 3)

        fire_idx(0, 0)
        fire_idx(1, 1)
        wait_idx(0)
        fire_g(0, 0)
        step(0, 0, True, True, False)
        step(1, 1, True, True, True)

        def six(j, carry):
            ci0 = 6 * j + 2
            for k in range(6):
                step(ci0 + k, 2 + k, True, True, True)
            return carry

        lax.fori_loop(0, (n - 5) // 6, six, 0)

        step(n - 3, n - 3, True, True, True)
        step(n - 2, n - 2, True, False, True)
        step(n - 1, n - 1, False, False, True)
        wait_s((n - 1) % 3, (n - 1) % 2)
        wait_deg((n - 1) % 3)

        plsc.subcore_barrier()
        pltpu.sync_copy(agg_sh.at[sl], agg_out.at[c, sl])
        if want_deg:
            pltpu.sync_copy(deg_sh.at[sl], deg_out.at[c, sl])

    return pl.kernel(body, out_type=out_type, mesh=_mesh,
                     scratch_types=scratch, compiler_params=_sc_params)


_sc_seg_sum_deg = _make_seg_sum(True)
_sc_seg_sum = _make_seg_sum(False)


@functools.partial(
    pl.kernel,
    out_type=jax.ShapeDtypeStruct((EQ, 64), jnp.float32),
    mesh=_mesh,
    scratch_types=[
        pltpu.VMEM((K,), jnp.int32), pltpu.VMEM((K,), jnp.int32),
        pltpu.VMEM((K,), jnp.int32), pltpu.VMEM((K,), jnp.int32),
        pltpu.VMEM((K,), jnp.int32), pltpu.VMEM((K,), jnp.int32),
        pltpu.VMEM((K, 64), jnp.float32), pltpu.VMEM((K, 64), jnp.float32),
        pltpu.VMEM((K, 64), jnp.float32), pltpu.VMEM((K, 64), jnp.float32),
        pltpu.SemaphoreType.DMA, pltpu.SemaphoreType.DMA,
        pltpu.SemaphoreType.DMA, pltpu.SemaphoreType.DMA,
        pltpu.SemaphoreType.DMA, pltpu.SemaphoreType.DMA,
        pltpu.SemaphoreType.DMA, pltpu.SemaphoreType.DMA,
        pltpu.SemaphoreType.DMA,
    ],
    compiler_params=_sc_params,
)
def _sc_query_gather(a_hbm, b_hbm, qsrc_hbm, qdst_hbm,
                     q_out,
                     si0, di0, si1, di1, si2, di2, a0, b0, a1, b1,
                     ga0, gb0, ga1, gb1, o0, o1, is0, is1, is2):
    sidx = [si0, si1, si2]
    didx = [di0, di1, di2]
    av = [a0, a1]
    bv = [b0, b1]
    gas = [ga0, ga1]
    gbs = [gb0, gb1]
    osem = [o0, o1]
    isem = [is0, is1, is2]
    c = lax.axis_index("c")
    s = lax.axis_index("s")
    w = s * NC + c
    base0 = w * (EQ // NW)
    n = EQ // NW // K

    def fire_idx(i, p):
        b = base0 + i * K
        pltpu.async_copy(qsrc_hbm.at[pl.ds(b, K)], sidx[p], isem[p])
        pltpu.async_copy(qdst_hbm.at[pl.ds(b, K)], didx[p], isem[p])

    def wait_idx(p):
        pltpu.make_async_copy(qsrc_hbm.at[pl.ds(base0, K)], sidx[p],
                              isem[p]).wait()
        pltpu.make_async_copy(qdst_hbm.at[pl.ds(base0, K)], didx[p],
                              isem[p]).wait()

    def fire_g(p, b):
        pltpu.async_copy(a_hbm.at[sidx[p]], av[b], gas[b])
        pltpu.async_copy(b_hbm.at[didx[p]], bv[b], gbs[b])

    def wait_g(p, b):
        pltpu.make_async_copy(a_hbm.at[sidx[p]], av[b], gas[b]).wait()
        pltpu.make_async_copy(b_hbm.at[didx[p]], bv[b], gbs[b]).wait()

    def add_rows(b):
        a_v, b_v = av[b], bv[b]

        def row(r, carry):
            for l in range(4):
                slc = pl.ds(l * 16, 16)
                a_v[r, slc] = a_v[r, slc] + b_v[r, slc]
            return carry
        lax.fori_loop(0, K, row, 0)

    def fire_out(ci, b):
        pltpu.async_copy(av[b], q_out.at[pl.ds(base0 + ci * K, K)], osem[b])

    def wait_out(b):
        pltpu.make_async_copy(av[b], q_out.at[pl.ds(base0, K)],
                              osem[b]).wait()

    def step(ci, pk, fire_next_g, fire_next_idx, drain_prev):
        p, b = pk % 3, pk % 2
        wait_g(p, b)
        if drain_prev:
            wait_out((pk - 1) % 2)
        if fire_next_g:
            wait_idx((pk + 1) % 3)
            fire_g((pk + 1) % 3, (pk + 1) % 2)
        if fire_next_idx:
            fire_idx(ci + 2, (pk + 2) % 3)
        add_rows(b)
        fire_out(ci, b)

    fire_idx(0, 0)
    fire_idx(1, 1)
    wait_idx(0)
    fire_g(0, 0)
    step(0, 0, True, True, False)
    step(1, 1, True, True, True)

    def six(j, carry):
        ci0 = 6 * j + 2
        for k in range(6):
            step(ci0 + k, 2 + k, True, True, True)
        return carry

    lax.fori_loop(0, (n - 5) // 6, six, 0)

    step(n - 3, n - 3, True, True, True)
    step(n - 2, n - 2, True, False, True)
    step(n - 1, n - 1, False, False, True)
    wait_out((n - 1) % 2)


BN = 1000


def _tc1_body(aggp_ref, degp_ref, x_ref, wl_ref, bl_ref, wr_ref, h1_ref):
    agg = aggp_ref[0] + aggp_ref[1]
    deg = degp_ref[0, :, 0] + degp_ref[1, :, 0]
    rdeg = 1.0 / jnp.maximum(deg, 1.0)
    mean = agg * rdeg[:, None]
    out = (jnp.dot(mean, wl_ref[...], preferred_element_type=jnp.float32)
           + bl_ref[...]
           + jnp.dot(x_ref[...], wr_ref[...], preferred_element_type=jnp.float32))
    h1_ref[...] = jnp.maximum(out, 0.0)


def _tc1(aggp, degp, x, wl, bl, wr):
    return pl.pallas_call(
        _tc1_body,
        grid=(NP // BN,),
        in_specs=[
            pl.BlockSpec((NC, BN, H), lambda i: (0, i, 0)),
            pl.BlockSpec((NC, BN, 16), lambda i: (0, i, 0)),
            pl.BlockSpec((BN, H), lambda i: (i, 0)),
            pl.BlockSpec((H, H), lambda i: (0, 0)),
            pl.BlockSpec((1, H), lambda i: (0, 0)),
            pl.BlockSpec((H, H), lambda i: (0, 0)),
        ],
        out_specs=pl.BlockSpec((BN, H), lambda i: (i, 0)),
        out_shape=jax.ShapeDtypeStruct((NP, H), jnp.float32),
    )(aggp, degp, x, wl, bl, wr)


def _tc2_body(aggp_ref, degp_ref, h1_ref, wl_ref, bl_ref, wr_ref, wsd_ref,
              a_ref, b_ref):
    agg = aggp_ref[0] + aggp_ref[1]
    deg = degp_ref[0, :, 0] + degp_ref[1, :, 0]
    rdeg = 1.0 / jnp.maximum(deg, 1.0)
    mean = agg * rdeg[:, None]
    h2 = (jnp.dot(mean, wl_ref[...], preferred_element_type=jnp.float32)
          + bl_ref[...]
          + jnp.dot(h1_ref[...], wr_ref[...], preferred_element_type=jnp.float32))
    ab = jnp.dot(h2, wsd_ref[...], preferred_element_type=jnp.float32)
    a_ref[...] = ab[:, :64]
    b_ref[...] = ab[:, 64:]


def _tc2(aggp, degp, h1, wl, bl, wr, wsd):
    return pl.pallas_call(
        _tc2_body,
        grid=(NP // BN,),
        in_specs=[
            pl.BlockSpec((NC, BN, H), lambda i: (0, i, 0)),
            pl.BlockSpec((NC, BN, 16), lambda i: (0, i, 0)),
            pl.BlockSpec((BN, H), lambda i: (i, 0)),
            pl.BlockSpec((H, H), lambda i: (0, 0)),
            pl.BlockSpec((1, H), lambda i: (0, 0)),
            pl.BlockSpec((H, H), lambda i: (0, 0)),
            pl.BlockSpec((H, H), lambda i: (0, 0)),
        ],
        out_specs=[
            pl.BlockSpec((BN, 64), lambda i: (i, 0)),
            pl.BlockSpec((BN, 64), lambda i: (i, 0)),
        ],
        out_shape=[
            jax.ShapeDtypeStruct((NP, 64), jnp.float32),
            jax.ShapeDtypeStruct((NP, 64), jnp.float32),
        ],
    )(aggp, degp, h1, wl, bl, wr, wsd)


EH = EQ // 2
BQ = 16000
GQ = EH // BQ
BE = 2000
BR = 8


def _tc_mlp_body(q_ref, attr_ref, w2a_ref, bc1_ref, w2c2_ref, bc2_ref,
                 wc3_ref, bc3_ref, oute_ref, outo_ref):
    q = q_ref[...]
    attr = attr_ref[...]
    z1 = q + jnp.dot(attr, w2a_ref[...], preferred_element_type=jnp.float32) + bc1_ref[...]
    z1 = jnp.maximum(z1, 0.0)
    z2 = jnp.dot(z1, w2c2_ref[...], preferred_element_type=jnp.float32) + bc2_ref[...]
    z2 = jnp.maximum(z2, 0.0)
    ze = jnp.sum(z2[:, :32] * wc3_ref[...], axis=1) + bc3_ref[0, 0]
    zo = jnp.sum(z2[:, 32:] * wc3_ref[...], axis=1) + bc3_ref[0, 0]
    oute_ref[...] = ze.reshape(BR, BE)
    outo_ref[...] = zo.reshape(BR, BE)


def _tc_mlp(q2, attr2, w2a, bc1_2, w2c2, bc2_2, wc3, bc3):
    return pl.pallas_call(
        _tc_mlp_body,
        grid=(GQ,),
        in_specs=[
            pl.BlockSpec((BQ, 128), lambda i: (i, 0)),
            pl.BlockSpec((BQ, 2 * DE), lambda i: (i, 0)),
            pl.BlockSpec((2 * DE, 128), lambda i: (0, 0)),
            pl.BlockSpec((1, 128), lambda i: (0, 0)),
            pl.BlockSpec((128, 64), lambda i: (0, 0)),
            pl.BlockSpec((1, 64), lambda i: (0, 0)),
            pl.BlockSpec((1, 32), lambda i: (0, 0)),
            pl.BlockSpec((1, 1), lambda i: (0, 0)),
        ],
        out_specs=[
            pl.BlockSpec((BR, BE), lambda i: (i, 0)),
            pl.BlockSpec((BR, BE), lambda i: (i, 0)),
        ],
        out_shape=[
            jax.ShapeDtypeStruct((GQ * BR, BE), jnp.float32),
            jax.ShapeDtypeStruct((GQ * BR, BE), jnp.float32),
        ],
    )(q2, attr2, w2a, bc1_2, w2c2, bc2_2, wc3, bc3)


def kernel(x, message_edge_index, query_edge_index, query_edge_attr,
           W1l, b1l, W1r, W2l, b2l, W2r,
           Wc1, bc1, Wc2, bc2, Wc3, bc3):
    src = message_edge_index[0]
    dst = message_edge_index[1]
    qsrc = query_edge_index[0]
    qdst = query_edge_index[1]
    zrows = jnp.zeros((NP, H), jnp.float32)
    zdeg = jnp.zeros((NP, 16), jnp.float32)
    onesrow = jnp.zeros((K, 16), jnp.float32).at[:, 0].set(1.0)

    agg1p, degp = _sc_seg_sum_deg(x, src, dst, zrows, zdeg, onesrow)
    h1 = _tc1(agg1p, degp, x, W1l, b1l[None, :], W1r)

    agg2p, = _sc_seg_sum(h1, src, dst, zrows, zdeg, onesrow)
    Wsd = jnp.concatenate([Wc1[:H], Wc1[H:2 * H]], axis=1)
    a_tab, b_tab = _tc2(agg2p, degp, h1, W2l, b2l[None, :], W2r, Wsd)

    q = _sc_query_gather(a_tab, b_tab, qsrc, qdst)
    q2 = q.reshape(EH, 128)
    Wa = Wc1[2 * H:]
    w2a = jnp.zeros((2 * DE, 128), jnp.float32)
    w2a = w2a.at[:DE, :64].set(Wa).at[DE:, 64:].set(Wa)
    w2c2 = jnp.zeros((128, 64), jnp.float32)
    w2c2 = w2c2.at[:64, :32].set(Wc2).at[64:, 32:].set(Wc2)
    bc1_2 = jnp.concatenate([bc1, bc1])[None, :]
    bc2_2 = jnp.concatenate([bc2, bc2])[None, :]
    oute, outo = _tc_mlp(q2, query_edge_attr.reshape(EH, 2 * DE),
                         w2a, bc1_2, w2c2, bc2_2,
                         Wc3.reshape(1, 32), bc3.reshape(1, 1))
    out = jnp.stack([oute.reshape(EH), outo.reshape(EH)], axis=-1)
    return out.reshape(EQ)


# speedup vs baseline: 1.1140x; 1.0716x over previous
import functools

import jax
import jax.numpy as jnp
from jax import lax
from jax.experimental import pallas as pl
from jax.experimental.pallas import tpu as pltpu
from jax.experimental.pallas import tpu_sc as plsc

N = 10000
D = 128
H = 128
E = 320000
EQ = 320000
DE = 16

NP = 10000
NC = 2
NS = 16
NW = NC * NS
K = 80
ROWS_PER_S = NP // NS
NCHUNK = E // NW // K

_mesh = plsc.VectorSubcoreMesh(core_axis_name="c", subcore_axis_name="s")
_sc_params = pltpu.CompilerParams(use_tc_tiling_on_sc=False)


KSEG = 128
TK = (E // NW) % KSEG


def _make_seg_sum(want_deg):
    out_type = [jax.ShapeDtypeStruct((NC, NP, H), jnp.float32)]
    scratch = [
        pltpu.VMEM((KSEG,), jnp.int32), pltpu.VMEM((KSEG,), jnp.int32),
        pltpu.VMEM((KSEG,), jnp.int32), pltpu.VMEM((KSEG,), jnp.int32),
        pltpu.VMEM((KSEG,), jnp.int32), pltpu.VMEM((KSEG,), jnp.int32),
        pltpu.VMEM((KSEG, H), jnp.float32), pltpu.VMEM((KSEG, H), jnp.float32),
        pltpu.VMEM((TK,), jnp.int32), pltpu.VMEM((TK,), jnp.int32),
        pltpu.VMEM((TK, H), jnp.float32),
        pltpu.VMEM_SHARED((NP, H), jnp.float32),
        pltpu.SemaphoreType.DMA, pltpu.SemaphoreType.DMA,
        pltpu.SemaphoreType.DMA, pltpu.SemaphoreType.DMA,
        pltpu.SemaphoreType.DMA, pltpu.SemaphoreType.DMA,
        pltpu.SemaphoreType.DMA,
    ]
    if want_deg:
        out_type.append(jax.ShapeDtypeStruct((NC, NP, 16), jnp.float32))
        scratch += [
            pltpu.VMEM((KSEG, 16), jnp.float32),
            pltpu.VMEM_SHARED((NP, 16), jnp.float32),
            pltpu.SemaphoreType.DMA,
        ]

    def body(x_hbm, src_hbm, dst_hbm, zrows_hbm, zdeg_hbm, onesrow_hbm,
             *refs):
        if want_deg:
            (agg_out, deg_out,
             si0, di0, si1, di1, si2, di2, rows0, rows1,
             tsi, tdi, trows, agg_sh,
             g0, g1, s0, s1, i0, i1, i2, ones_v, deg_sh, dsem) = refs
        else:
            (agg_out,
             si0, di0, si1, di1, si2, di2, rows0, rows1,
             tsi, tdi, trows, agg_sh,
             g0, g1, s0, s1, i0, i1, i2) = refs
        sidx = [si0, si1, si2]
        didx = [di0, di1, di2]
        rows = [rows0, rows1]
        gsem = [g0, g1]
        ssem = [s0, s1]
        isem = [i0, i1, i2]
        c = lax.axis_index("c")
        s = lax.axis_index("s")
        w = s * NC + c
        base0 = w * (E // NW)
        K = KSEG
        n = E // NW // K

        sl = pl.ds(s * ROWS_PER_S, ROWS_PER_S)
        pltpu.sync_copy(zrows_hbm.at[sl], agg_sh.at[sl])
        if want_deg:
            pltpu.sync_copy(zdeg_hbm.at[sl], deg_sh.at[sl])
            pltpu.sync_copy(onesrow_hbm, ones_v)
        plsc.subcore_barrier()

        def fire_idx(i, p, base=None):
            b = base0 + i * K if base is None else base
            pltpu.async_copy(src_hbm.at[pl.ds(b, K)], sidx[p], isem[p])
            pltpu.async_copy(dst_hbm.at[pl.ds(b, K)], didx[p], isem[p])

        def wait_idx(p):
            pltpu.make_async_copy(src_hbm.at[pl.ds(base0, K)], sidx[p],
                                  isem[p]).wait()
            pltpu.make_async_copy(dst_hbm.at[pl.ds(base0, K)], didx[p],
                                  isem[p]).wait()

        def fire_g(p, b):
            pltpu.async_copy(x_hbm.at[sidx[p]], rows[b], gsem[b])

        def wait_g(p, b):
            pltpu.make_async_copy(x_hbm.at[sidx[p]], rows[b], gsem[b]).wait()

        def fire_s(p, b):
            pltpu.async_copy(rows[b], agg_sh.at[didx[p]], ssem[b], add=True)

        def wait_s(p, b):
            pltpu.make_async_copy(rows[b], agg_sh.at[didx[p]], ssem[b]).wait()

        def fire_deg(p):
            if want_deg:
                pltpu.async_copy(ones_v, deg_sh.at[didx[p]], dsem, add=True)

        def wait_deg(p):
            if want_deg:
                pltpu.make_async_copy(ones_v, deg_sh.at[didx[p]], dsem).wait()

        def step(ci, pk, fire_next_g, fire_next_idx, drain_prev):
            p, b = pk % 3, pk % 2
            wait_g(p, b)
            fire_s(p, b)
            fire_deg(p)
            if drain_prev:
                wait_s((pk - 1) % 3, (pk - 1) % 2)
                wait_deg((pk - 1) % 3)
            if fire_next_g:
                wait_idx((pk + 1) % 3)
                fire_g((pk + 1) % 3, (pk + 1) % 2)
            if fire_next_idx:
                fire_idx(ci + 2, (pk + 2) % 3)

        fire_idx(0, 0)
        fire_idx(1, 1)
        wait_idx(0)
        fire_g(0, 0)
        step(0, 0, True, True, False)
        step(1, 1, True, True, True)

        def six(j, carry):
            ci0 = 6 * j + 2
            for k in range(6):
                step(ci0 + k, 2 + k, True, True, True)
            return carry

        lax.fori_loop(0, (n - 5) // 6, six, 0)
        for ci in range(2 + 6 * ((n - 5) // 6), n):
            step(ci, ci, ci + 1 < n, ci + 2 < n, True)
        wait_s((n - 1) % 3, (n - 1) % 2)
        wait_deg((n - 1) % 3)

        if TK:
            tb = base0 + n * K
            pltpu.sync_copy(src_hbm.at[pl.ds(tb, TK)], tsi)
            pltpu.sync_copy(dst_hbm.at[pl.ds(tb, TK)], tdi)
            pltpu.async_copy(x_hbm.at[tsi], trows, g0).wait()
            cp = pltpu.async_copy(trows, agg_sh.at[tdi], s0, add=True)
            if want_deg:
                pltpu.async_copy(ones_v.at[pl.ds(0, TK)], deg_sh.at[tdi],
                                 dsem, add=True).wait()
            cp.wait()

        plsc.subcore_barrier()
        pltpu.sync_copy(agg_sh.at[sl], agg_out.at[c, sl])
        if want_deg:
            pltpu.sync_copy(deg_sh.at[sl], deg_out.at[c, sl])

    return pl.kernel(body, out_type=out_type, mesh=_mesh,
                     scratch_types=scratch, compiler_params=_sc_params)


_sc_seg_sum_deg = _make_seg_sum(True)
_sc_seg_sum = _make_seg_sum(False)


@functools.partial(
    pl.kernel,
    out_type=jax.ShapeDtypeStruct((EQ, 64), jnp.float32),
    mesh=_mesh,
    scratch_types=[
        pltpu.VMEM((K,), jnp.int32), pltpu.VMEM((K,), jnp.int32),
        pltpu.VMEM((K,), jnp.int32), pltpu.VMEM((K,), jnp.int32),
        pltpu.VMEM((K,), jnp.int32), pltpu.VMEM((K,), jnp.int32),
        pltpu.VMEM((K, 64), jnp.float32), pltpu.VMEM((K, 64), jnp.float32),
        pltpu.VMEM((K, 64), jnp.float32), pltpu.VMEM((K, 64), jnp.float32),
        pltpu.SemaphoreType.DMA, pltpu.SemaphoreType.DMA,
        pltpu.SemaphoreType.DMA, pltpu.SemaphoreType.DMA,
        pltpu.SemaphoreType.DMA, pltpu.SemaphoreType.DMA,
        pltpu.SemaphoreType.DMA, pltpu.SemaphoreType.DMA,
        pltpu.SemaphoreType.DMA,
    ],
    compiler_params=_sc_params,
)
def _sc_query_gather(a_hbm, b_hbm, qsrc_hbm, qdst_hbm,
                     q_out,
                     si0, di0, si1, di1, si2, di2, a0, b0, a1, b1,
                     ga0, gb0, ga1, gb1, o0, o1, is0, is1, is2):
    sidx = [si0, si1, si2]
    didx = [di0, di1, di2]
    av = [a0, a1]
    bv = [b0, b1]
    gas = [ga0, ga1]
    gbs = [gb0, gb1]
    osem = [o0, o1]
    isem = [is0, is1, is2]
    c = lax.axis_index("c")
    s = lax.axis_index("s")
    w = s * NC + c
    base0 = w * (EQ // NW)
    n = EQ // NW // K

    def fire_idx(i, p):
        b = base0 + i * K
        pltpu.async_copy(qsrc_hbm.at[pl.ds(b, K)], sidx[p], isem[p])
        pltpu.async_copy(qdst_hbm.at[pl.ds(b, K)], didx[p], isem[p])

    def wait_idx(p):
        pltpu.make_async_copy(qsrc_hbm.at[pl.ds(base0, K)], sidx[p],
                              isem[p]).wait()
        pltpu.make_async_copy(qdst_hbm.at[pl.ds(base0, K)], didx[p],
                              isem[p]).wait()

    def fire_g(p, b):
        pltpu.async_copy(a_hbm.at[sidx[p]], av[b], gas[b])
        pltpu.async_copy(b_hbm.at[didx[p]], bv[b], gbs[b])

    def wait_g(p, b):
        pltpu.make_async_copy(a_hbm.at[sidx[p]], av[b], gas[b]).wait()
        pltpu.make_async_copy(b_hbm.at[didx[p]], bv[b], gbs[b]).wait()

    def add_rows(b):
        a_v, b_v = av[b], bv[b]

        def row(r, carry):
            for l in range(4):
                slc = pl.ds(l * 16, 16)
                a_v[r, slc] = a_v[r, slc] + b_v[r, slc]
            return carry
        lax.fori_loop(0, K, row, 0)

    def fire_out(ci, b):
        pltpu.async_copy(av[b], q_out.at[pl.ds(base0 + ci * K, K)], osem[b])

    def wait_out(b):
        pltpu.make_async_copy(av[b], q_out.at[pl.ds(base0, K)],
                              osem[b]).wait()

    def step(ci, pk, fire_next_g, fire_next_idx, drain_prev):
        p, b = pk % 3, pk % 2
        wait_g(p, b)
        if drain_prev:
            wait_out((pk - 1) % 2)
        if fire_next_g:
            wait_idx((pk + 1) % 3)
            fire_g((pk + 1) % 3, (pk + 1) % 2)
        if fire_next_idx:
            fire_idx(ci + 2, (pk + 2) % 3)
        add_rows(b)
        fire_out(ci, b)

    fire_idx(0, 0)
    fire_idx(1, 1)
    wait_idx(0)
    fire_g(0, 0)
    step(0, 0, True, True, False)
    step(1, 1, True, True, True)

    def six(j, carry):
        ci0 = 6 * j + 2
        for k in range(6):
            step(ci0 + k, 2 + k, True, True, True)
        return carry

    lax.fori_loop(0, (n - 5) // 6, six, 0)

    step(n - 3, n - 3, True, True, True)
    step(n - 2, n - 2, True, False, True)
    step(n - 1, n - 1, False, False, True)
    wait_out((n - 1) % 2)


BN = 1000


def _tc1_body(aggp_ref, degp_ref, x_ref, wl_ref, bl_ref, wr_ref, h1_ref):
    agg = aggp_ref[0] + aggp_ref[1]
    deg = degp_ref[0, :, 0] + degp_ref[1, :, 0]
    rdeg = 1.0 / jnp.maximum(deg, 1.0)
    mean = agg * rdeg[:, None]
    out = (jnp.dot(mean, wl_ref[...], preferred_element_type=jnp.float32)
           + bl_ref[...]
           + jnp.dot(x_ref[...], wr_ref[...], preferred_element_type=jnp.float32))
    h1_ref[...] = jnp.maximum(out, 0.0)


def _tc1(aggp, degp, x, wl, bl, wr):
    return pl.pallas_call(
        _tc1_body,
        grid=(NP // BN,),
        in_specs=[
            pl.BlockSpec((NC, BN, H), lambda i: (0, i, 0)),
            pl.BlockSpec((NC, BN, 16), lambda i: (0, i, 0)),
            pl.BlockSpec((BN, H), lambda i: (i, 0)),
            pl.BlockSpec((H, H), lambda i: (0, 0)),
            pl.BlockSpec((1, H), lambda i: (0, 0)),
            pl.BlockSpec((H, H), lambda i: (0, 0)),
        ],
        out_specs=pl.BlockSpec((BN, H), lambda i: (i, 0)),
        out_shape=jax.ShapeDtypeStruct((NP, H), jnp.float32),
    )(aggp, degp, x, wl, bl, wr)


def _tc2_body(aggp_ref, degp_ref, h1_ref, wl_ref, bl_ref, wr_ref, wsd_ref,
              a_ref, b_ref):
    agg = aggp_ref[0] + aggp_ref[1]
    deg = degp_ref[0, :, 0] + degp_ref[1, :, 0]
    rdeg = 1.0 / jnp.maximum(deg, 1.0)
    mean = agg * rdeg[:, None]
    h2 = (jnp.dot(mean, wl_ref[...], preferred_element_type=jnp.float32)
          + bl_ref[...]
          + jnp.dot(h1_ref[...], wr_ref[...], preferred_element_type=jnp.float32))
    ab = jnp.dot(h2, wsd_ref[...], preferred_element_type=jnp.float32)
    a_ref[...] = ab[:, :64]
    b_ref[...] = ab[:, 64:]


def _tc2(aggp, degp, h1, wl, bl, wr, wsd):
    return pl.pallas_call(
        _tc2_body,
        grid=(NP // BN,),
        in_specs=[
            pl.BlockSpec((NC, BN, H), lambda i: (0, i, 0)),
            pl.BlockSpec((NC, BN, 16), lambda i: (0, i, 0)),
            pl.BlockSpec((BN, H), lambda i: (i, 0)),
            pl.BlockSpec((H, H), lambda i: (0, 0)),
            pl.BlockSpec((1, H), lambda i: (0, 0)),
            pl.BlockSpec((H, H), lambda i: (0, 0)),
            pl.BlockSpec((H, H), lambda i: (0, 0)),
        ],
        out_specs=[
            pl.BlockSpec((BN, 64), lambda i: (i, 0)),
            pl.BlockSpec((BN, 64), lambda i: (i, 0)),
        ],
        out_shape=[
            jax.ShapeDtypeStruct((NP, 64), jnp.float32),
            jax.ShapeDtypeStruct((NP, 64), jnp.float32),
        ],
    )(aggp, degp, h1, wl, bl, wr, wsd)


EH = EQ // 2
BQ = 16000
GQ = EH // BQ
BE = 2000
BR = 8


def _tc_mlp_body(q_ref, attr_ref, w2a_ref, bc1_ref, w2c2_ref, bc2_ref,
                 wc3_ref, bc3_ref, oute_ref, outo_ref):
    q = q_ref[...]
    attr = attr_ref[...]
    z1 = q + jnp.dot(attr, w2a_ref[...], preferred_element_type=jnp.float32) + bc1_ref[...]
    z1 = jnp.maximum(z1, 0.0)
    z2 = jnp.dot(z1, w2c2_ref[...], preferred_element_type=jnp.float32) + bc2_ref[...]
    z2 = jnp.maximum(z2, 0.0)
    ze = jnp.sum(z2[:, :32] * wc3_ref[...], axis=1) + bc3_ref[0, 0]
    zo = jnp.sum(z2[:, 32:] * wc3_ref[...], axis=1) + bc3_ref[0, 0]
    oute_ref[...] = ze.reshape(BR, BE)
    outo_ref[...] = zo.reshape(BR, BE)


def _tc_mlp(q2, attr2, w2a, bc1_2, w2c2, bc2_2, wc3, bc3):
    return pl.pallas_call(
        _tc_mlp_body,
        grid=(GQ,),
        in_specs=[
            pl.BlockSpec((BQ, 128), lambda i: (i, 0)),
            pl.BlockSpec((BQ, 2 * DE), lambda i: (i, 0)),
            pl.BlockSpec((2 * DE, 128), lambda i: (0, 0)),
            pl.BlockSpec((1, 128), lambda i: (0, 0)),
            pl.BlockSpec((128, 64), lambda i: (0, 0)),
            pl.BlockSpec((1, 64), lambda i: (0, 0)),
            pl.BlockSpec((1, 32), lambda i: (0, 0)),
            pl.BlockSpec((1, 1), lambda i: (0, 0)),
        ],
        out_specs=[
            pl.BlockSpec((BR, BE), lambda i: (i, 0)),
            pl.BlockSpec((BR, BE), lambda i: (i, 0)),
        ],
        out_shape=[
            jax.ShapeDtypeStruct((GQ * BR, BE), jnp.float32),
            jax.ShapeDtypeStruct((GQ * BR, BE), jnp.float32),
        ],
    )(q2, attr2, w2a, bc1_2, w2c2, bc2_2, wc3, bc3)


def kernel(x, message_edge_index, query_edge_index, query_edge_attr,
           W1l, b1l, W1r, W2l, b2l, W2r,
           Wc1, bc1, Wc2, bc2, Wc3, bc3):
    src = message_edge_index[0]
    dst = message_edge_index[1]
    qsrc = query_edge_index[0]
    qdst = query_edge_index[1]
    zrows = jnp.zeros((NP, H), jnp.float32)
    zdeg = jnp.zeros((NP, 16), jnp.float32)
    onesrow = jnp.zeros((KSEG, 16), jnp.float32).at[:, 0].set(1.0)

    agg1p, degp = _sc_seg_sum_deg(x, src, dst, zrows, zdeg, onesrow)
    h1 = _tc1(agg1p, degp, x, W1l, b1l[None, :], W1r)

    agg2p, = _sc_seg_sum(h1, src, dst, zrows, zdeg, onesrow)
    Wsd = jnp.concatenate([Wc1[:H], Wc1[H:2 * H]], axis=1)
    a_tab, b_tab = _tc2(agg2p, degp, h1, W2l, b2l[None, :], W2r, Wsd)

    q = _sc_query_gather(a_tab, b_tab, qsrc, qdst)
    q2 = q.reshape(EH, 128)
    Wa = Wc1[2 * H:]
    w2a = jnp.zeros((2 * DE, 128), jnp.float32)
    w2a = w2a.at[:DE, :64].set(Wa).at[DE:, 64:].set(Wa)
    w2c2 = jnp.zeros((128, 64), jnp.float32)
    w2c2 = w2c2.at[:64, :32].set(Wc2).at[64:, 32:].set(Wc2)
    bc1_2 = jnp.concatenate([bc1, bc1])[None, :]
    bc2_2 = jnp.concatenate([bc2, bc2])[None, :]
    oute, outo = _tc_mlp(q2, query_edge_attr.reshape(EH, 2 * DE),
                         w2a, bc1_2, w2c2, bc2_2,
                         Wc3.reshape(1, 32), bc3.reshape(1, 1))
    out = jnp.stack([oute.reshape(EH), outo.reshape(EH)], axis=-1)
    return out.reshape(EQ)


# speedup vs baseline: 1.1286x; 1.0131x over previous
import functools

import jax
import jax.numpy as jnp
from jax import lax
from jax.experimental import pallas as pl
from jax.experimental.pallas import tpu as pltpu
from jax.experimental.pallas import tpu_sc as plsc

N = 10000
D = 128
H = 128
E = 320000
EQ = 320000
DE = 16

NP = 10000
NC = 2
NS = 16
NW = NC * NS
K = 80
ROWS_PER_S = NP // NS
NCHUNK = E // NW // K

_mesh = plsc.VectorSubcoreMesh(core_axis_name="c", subcore_axis_name="s")
_sc_params = pltpu.CompilerParams(use_tc_tiling_on_sc=False)


KSEG = 128
TK = (E // NW) % KSEG


def _make_seg_sum(want_deg):
    out_type = [jax.ShapeDtypeStruct((NC, NP, H), jnp.float32)]
    scratch = [
        pltpu.VMEM((KSEG,), jnp.int32), pltpu.VMEM((KSEG,), jnp.int32),
        pltpu.VMEM((KSEG,), jnp.int32), pltpu.VMEM((KSEG,), jnp.int32),
        pltpu.VMEM((KSEG,), jnp.int32), pltpu.VMEM((KSEG,), jnp.int32),
        pltpu.VMEM((KSEG, H), jnp.float32), pltpu.VMEM((KSEG, H), jnp.float32),
        pltpu.VMEM((TK,), jnp.int32), pltpu.VMEM((TK,), jnp.int32),
        pltpu.VMEM((TK, H), jnp.float32),
        pltpu.VMEM_SHARED((NP, H), jnp.float32),
        pltpu.SemaphoreType.DMA, pltpu.SemaphoreType.DMA,
        pltpu.SemaphoreType.DMA, pltpu.SemaphoreType.DMA,
        pltpu.SemaphoreType.DMA, pltpu.SemaphoreType.DMA,
        pltpu.SemaphoreType.DMA,
    ]
    if want_deg:
        out_type.append(jax.ShapeDtypeStruct((NC, NP, 16), jnp.float32))
        scratch += [
            pltpu.VMEM((KSEG, 16), jnp.float32),
            pltpu.VMEM_SHARED((NP, 16), jnp.float32),
            pltpu.SemaphoreType.DMA,
        ]

    def body(x_hbm, src_hbm, dst_hbm, zrows_hbm, zdeg_hbm, onesrow_hbm,
             *refs):
        if want_deg:
            (agg_out, deg_out,
             si0, di0, si1, di1, si2, di2, rows0, rows1,
             tsi, tdi, trows, agg_sh,
             g0, g1, s0, s1, i0, i1, i2, ones_v, deg_sh, dsem) = refs
        else:
            (agg_out,
             si0, di0, si1, di1, si2, di2, rows0, rows1,
             tsi, tdi, trows, agg_sh,
             g0, g1, s0, s1, i0, i1, i2) = refs
        sidx = [si0, si1, si2]
        didx = [di0, di1, di2]
        rows = [rows0, rows1]
        gsem = [g0, g1]
        ssem = [s0, s1]
        isem = [i0, i1, i2]
        c = lax.axis_index("c")
        s = lax.axis_index("s")
        w = s * NC + c
        base0 = w * (E // NW)
        K = KSEG
        n = E // NW // K

        sl = pl.ds(s * ROWS_PER_S, ROWS_PER_S)
        pltpu.sync_copy(zrows_hbm.at[sl], agg_sh.at[sl])
        if want_deg:
            pltpu.sync_copy(zdeg_hbm.at[sl], deg_sh.at[sl])
            pltpu.sync_copy(onesrow_hbm, ones_v)
        plsc.subcore_barrier()

        def fire_idx(i, p, base=None):
            b = base0 + i * K if base is None else base
            pltpu.async_copy(src_hbm.at[pl.ds(b, K)], sidx[p], isem[p])
            pltpu.async_copy(dst_hbm.at[pl.ds(b, K)], didx[p], isem[p])

        def wait_idx(p):
            pltpu.make_async_copy(src_hbm.at[pl.ds(base0, K)], sidx[p],
                                  isem[p]).wait()
            pltpu.make_async_copy(dst_hbm.at[pl.ds(base0, K)], didx[p],
                                  isem[p]).wait()

        def fire_g(p, b):
            pltpu.async_copy(x_hbm.at[sidx[p]], rows[b], gsem[b])

        def wait_g(p, b):
            pltpu.make_async_copy(x_hbm.at[sidx[p]], rows[b], gsem[b]).wait()

        def fire_s(p, b):
            pltpu.async_copy(rows[b], agg_sh.at[didx[p]], ssem[b], add=True)

        def wait_s(p, b):
            pltpu.make_async_copy(rows[b], agg_sh.at[didx[p]], ssem[b]).wait()

        def fire_deg(p):
            if want_deg:
                pltpu.async_copy(ones_v, deg_sh.at[didx[p]], dsem, add=True)

        def wait_deg(p):
            if want_deg:
                pltpu.make_async_copy(ones_v, deg_sh.at[didx[p]], dsem).wait()

        def step(ci, pk, fire_next_g, fire_next_idx, drain_prev):
            p, b = pk % 3, pk % 2
            wait_g(p, b)
            fire_s(p, b)
            fire_deg(p)
            if drain_prev:
                wait_s((pk - 1) % 3, (pk - 1) % 2)
                wait_deg((pk - 1) % 3)
            if fire_next_g:
                wait_idx((pk + 1) % 3)
                fire_g((pk + 1) % 3, (pk + 1) % 2)
            if fire_next_idx:
                fire_idx(ci + 2, (pk + 2) % 3)

        fire_idx(0, 0)
        fire_idx(1, 1)
        wait_idx(0)
        fire_g(0, 0)
        step(0, 0, True, True, False)
        step(1, 1, True, True, True)

        def six(j, carry):
            ci0 = 6 * j + 2
            for k in range(6):
                step(ci0 + k, 2 + k, True, True, True)
            return carry

        lax.fori_loop(0, (n - 5) // 6, six, 0)
        for ci in range(2 + 6 * ((n - 5) // 6), n):
            step(ci, ci, ci + 1 < n, ci + 2 < n, True)
        wait_s((n - 1) % 3, (n - 1) % 2)
        wait_deg((n - 1) % 3)

        if TK:
            tb = base0 + n * K
            pltpu.sync_copy(src_hbm.at[pl.ds(tb, TK)], tsi)
            pltpu.sync_copy(dst_hbm.at[pl.ds(tb, TK)], tdi)
            pltpu.async_copy(x_hbm.at[tsi], trows, g0).wait()
            cp = pltpu.async_copy(trows, agg_sh.at[tdi], s0, add=True)
            if want_deg:
                pltpu.async_copy(ones_v.at[pl.ds(0, TK)], deg_sh.at[tdi],
                                 dsem, add=True).wait()
            cp.wait()

        plsc.subcore_barrier()
        pltpu.sync_copy(agg_sh.at[sl], agg_out.at[c, sl])
        if want_deg:
            pltpu.sync_copy(deg_sh.at[sl], deg_out.at[c, sl])

    return pl.kernel(body, out_type=out_type, mesh=_mesh,
                     scratch_types=scratch, compiler_params=_sc_params)


_sc_seg_sum_deg = _make_seg_sum(True)
_sc_seg_sum = _make_seg_sum(False)


@functools.partial(
    pl.kernel,
    out_type=jax.ShapeDtypeStruct((EQ, 64), jnp.float32),
    mesh=_mesh,
    scratch_types=[
        pltpu.VMEM((KSEG,), jnp.int32), pltpu.VMEM((KSEG,), jnp.int32),
        pltpu.VMEM((KSEG,), jnp.int32), pltpu.VMEM((KSEG,), jnp.int32),
        pltpu.VMEM((KSEG,), jnp.int32), pltpu.VMEM((KSEG,), jnp.int32),
        pltpu.VMEM((KSEG, 64), jnp.float32), pltpu.VMEM((KSEG, 64), jnp.float32),
        pltpu.VMEM((KSEG, 64), jnp.float32), pltpu.VMEM((KSEG, 64), jnp.float32),
        pltpu.VMEM((TK,), jnp.int32), pltpu.VMEM((TK,), jnp.int32),
        pltpu.VMEM((TK, 64), jnp.float32), pltpu.VMEM((TK, 64), jnp.float32),
        pltpu.SemaphoreType.DMA, pltpu.SemaphoreType.DMA,
        pltpu.SemaphoreType.DMA, pltpu.SemaphoreType.DMA,
        pltpu.SemaphoreType.DMA, pltpu.SemaphoreType.DMA,
        pltpu.SemaphoreType.DMA, pltpu.SemaphoreType.DMA,
        pltpu.SemaphoreType.DMA,
    ],
    compiler_params=_sc_params,
)
def _sc_query_gather(a_hbm, b_hbm, qsrc_hbm, qdst_hbm,
                     q_out,
                     si0, di0, si1, di1, si2, di2, a0, b0, a1, b1,
                     tsi, tdi, ta_v, tb_v,
                     ga0, gb0, ga1, gb1, o0, o1, is0, is1, is2):
    sidx = [si0, si1, si2]
    didx = [di0, di1, di2]
    av = [a0, a1]
    bv = [b0, b1]
    gas = [ga0, ga1]
    gbs = [gb0, gb1]
    osem = [o0, o1]
    isem = [is0, is1, is2]
    c = lax.axis_index("c")
    s = lax.axis_index("s")
    w = s * NC + c
    base0 = w * (EQ // NW)
    K = KSEG
    n = EQ // NW // K

    def fire_idx(i, p):
        b = base0 + i * K
        pltpu.async_copy(qsrc_hbm.at[pl.ds(b, K)], sidx[p], isem[p])
        pltpu.async_copy(qdst_hbm.at[pl.ds(b, K)], didx[p], isem[p])

    def wait_idx(p):
        pltpu.make_async_copy(qsrc_hbm.at[pl.ds(base0, K)], sidx[p],
                              isem[p]).wait()
        pltpu.make_async_copy(qdst_hbm.at[pl.ds(base0, K)], didx[p],
                              isem[p]).wait()

    def fire_g(p, b):
        pltpu.async_copy(a_hbm.at[sidx[p]], av[b], gas[b])
        pltpu.async_copy(b_hbm.at[didx[p]], bv[b], gbs[b])

    def wait_g(p, b):
        pltpu.make_async_copy(a_hbm.at[sidx[p]], av[b], gas[b]).wait()
        pltpu.make_async_copy(b_hbm.at[didx[p]], bv[b], gbs[b]).wait()

    def add_rows(b):
        a_v, b_v = av[b], bv[b]

        def row(r, carry):
            for l in range(4):
                slc = pl.ds(l * 16, 16)
                a_v[r, slc] = a_v[r, slc] + b_v[r, slc]
            return carry
        lax.fori_loop(0, K, row, 0)

    def fire_out(ci, b):
        pltpu.async_copy(av[b], q_out.at[pl.ds(base0 + ci * K, K)], osem[b])

    def wait_out(b):
        pltpu.make_async_copy(av[b], q_out.at[pl.ds(base0, K)],
                              osem[b]).wait()

    def step(ci, pk, fire_next_g, fire_next_idx, drain_prev):
        p, b = pk % 3, pk % 2
        wait_g(p, b)
        if drain_prev:
            wait_out((pk - 1) % 2)
        if fire_next_g:
            wait_idx((pk + 1) % 3)
            fire_g((pk + 1) % 3, (pk + 1) % 2)
        if fire_next_idx:
            fire_idx(ci + 2, (pk + 2) % 3)
        add_rows(b)
        fire_out(ci, b)

    fire_idx(0, 0)
    fire_idx(1, 1)
    wait_idx(0)
    fire_g(0, 0)
    step(0, 0, True, True, False)
    step(1, 1, True, True, True)

    def six(j, carry):
        ci0 = 6 * j + 2
        for k in range(6):
            step(ci0 + k, 2 + k, True, True, True)
        return carry

    lax.fori_loop(0, (n - 5) // 6, six, 0)
    for ci in range(2 + 6 * ((n - 5) // 6), n):
        step(ci, ci, ci + 1 < n, ci + 2 < n, True)
    wait_out((n - 1) % 2)

    if TK:
        tb = base0 + n * K
        pltpu.sync_copy(qsrc_hbm.at[pl.ds(tb, TK)], tsi)
        pltpu.sync_copy(qdst_hbm.at[pl.ds(tb, TK)], tdi)
        cpa = pltpu.async_copy(a_hbm.at[tsi], ta_v, ga0)
        cpb = pltpu.async_copy(b_hbm.at[tdi], tb_v, gb0)
        cpa.wait()
        cpb.wait()
        for r in range(TK):
            for l in range(4):
                slc = pl.ds(l * 16, 16)
                ta_v[r, slc] = ta_v[r, slc] + tb_v[r, slc]
        pltpu.sync_copy(ta_v, q_out.at[pl.ds(tb, TK)])


BN = 1000


def _tc1_body(aggp_ref, degp_ref, x_ref, wl_ref, bl_ref, wr_ref, h1_ref):
    agg = aggp_ref[0] + aggp_ref[1]
    deg = degp_ref[0, :, 0] + degp_ref[1, :, 0]
    rdeg = 1.0 / jnp.maximum(deg, 1.0)
    mean = agg * rdeg[:, None]
    out = (jnp.dot(mean, wl_ref[...], preferred_element_type=jnp.float32)
           + bl_ref[...]
           + jnp.dot(x_ref[...], wr_ref[...], preferred_element_type=jnp.float32))
    h1_ref[...] = jnp.maximum(out, 0.0)


def _tc1(aggp, degp, x, wl, bl, wr):
    return pl.pallas_call(
        _tc1_body,
        grid=(NP // BN,),
        in_specs=[
            pl.BlockSpec((NC, BN, H), lambda i: (0, i, 0)),
            pl.BlockSpec((NC, BN, 16), lambda i: (0, i, 0)),
            pl.BlockSpec((BN, H), lambda i: (i, 0)),
            pl.BlockSpec((H, H), lambda i: (0, 0)),
            pl.BlockSpec((1, H), lambda i: (0, 0)),
            pl.BlockSpec((H, H), lambda i: (0, 0)),
        ],
        out_specs=pl.BlockSpec((BN, H), lambda i: (i, 0)),
        out_shape=jax.ShapeDtypeStruct((NP, H), jnp.float32),
    )(aggp, degp, x, wl, bl, wr)


def _tc2_body(aggp_ref, degp_ref, h1_ref, wl_ref, bl_ref, wr_ref, wsd_ref,
              a_ref, b_ref):
    agg = aggp_ref[0] + aggp_ref[1]
    deg = degp_ref[0, :, 0] + degp_ref[1, :, 0]
    rdeg = 1.0 / jnp.maximum(deg, 1.0)
    mean = agg * rdeg[:, None]
    h2 = (jnp.dot(mean, wl_ref[...], preferred_element_type=jnp.float32)
          + bl_ref[...]
          + jnp.dot(h1_ref[...], wr_ref[...], preferred_element_type=jnp.float32))
    ab = jnp.dot(h2, wsd_ref[...], preferred_element_type=jnp.float32)
    a_ref[...] = ab[:, :64]
    b_ref[...] = ab[:, 64:]


def _tc2(aggp, degp, h1, wl, bl, wr, wsd):
    return pl.pallas_call(
        _tc2_body,
        grid=(NP // BN,),
        in_specs=[
            pl.BlockSpec((NC, BN, H), lambda i: (0, i, 0)),
            pl.BlockSpec((NC, BN, 16), lambda i: (0, i, 0)),
            pl.BlockSpec((BN, H), lambda i: (i, 0)),
            pl.BlockSpec((H, H), lambda i: (0, 0)),
            pl.BlockSpec((1, H), lambda i: (0, 0)),
            pl.BlockSpec((H, H), lambda i: (0, 0)),
            pl.BlockSpec((H, H), lambda i: (0, 0)),
        ],
        out_specs=[
            pl.BlockSpec((BN, 64), lambda i: (i, 0)),
            pl.BlockSpec((BN, 64), lambda i: (i, 0)),
        ],
        out_shape=[
            jax.ShapeDtypeStruct((NP, 64), jnp.float32),
            jax.ShapeDtypeStruct((NP, 64), jnp.float32),
        ],
    )(aggp, degp, h1, wl, bl, wr, wsd)


EH = EQ // 2
BQ = 16000
GQ = EH // BQ
BE = 2000
BR = 8


def _tc_mlp_body(q_ref, attr_ref, w2a_ref, bc1_ref, w2c2_ref, bc2_ref,
                 wc3_ref, bc3_ref, oute_ref, outo_ref):
    q = q_ref[...]
    attr = attr_ref[...]
    z1 = q + jnp.dot(attr, w2a_ref[...], preferred_element_type=jnp.float32) + bc1_ref[...]
    z1 = jnp.maximum(z1, 0.0)
    z2 = jnp.dot(z1, w2c2_ref[...], preferred_element_type=jnp.float32) + bc2_ref[...]
    z2 = jnp.maximum(z2, 0.0)
    ze = jnp.sum(z2[:, :32] * wc3_ref[...], axis=1) + bc3_ref[0, 0]
    zo = jnp.sum(z2[:, 32:] * wc3_ref[...], axis=1) + bc3_ref[0, 0]
    oute_ref[...] = ze.reshape(BR, BE)
    outo_ref[...] = zo.reshape(BR, BE)


def _tc_mlp(q2, attr2, w2a, bc1_2, w2c2, bc2_2, wc3, bc3):
    return pl.pallas_call(
        _tc_mlp_body,
        grid=(GQ,),
        in_specs=[
            pl.BlockSpec((BQ, 128), lambda i: (i, 0)),
            pl.BlockSpec((BQ, 2 * DE), lambda i: (i, 0)),
            pl.BlockSpec((2 * DE, 128), lambda i: (0, 0)),
            pl.BlockSpec((1, 128), lambda i: (0, 0)),
            pl.BlockSpec((128, 64), lambda i: (0, 0)),
            pl.BlockSpec((1, 64), lambda i: (0, 0)),
            pl.BlockSpec((1, 32), lambda i: (0, 0)),
            pl.BlockSpec((1, 1), lambda i: (0, 0)),
        ],
        out_specs=[
            pl.BlockSpec((BR, BE), lambda i: (i, 0)),
            pl.BlockSpec((BR, BE), lambda i: (i, 0)),
        ],
        out_shape=[
            jax.ShapeDtypeStruct((GQ * BR, BE), jnp.float32),
            jax.ShapeDtypeStruct((GQ * BR, BE), jnp.float32),
        ],
    )(q2, attr2, w2a, bc1_2, w2c2, bc2_2, wc3, bc3)


def kernel(x, message_edge_index, query_edge_index, query_edge_attr,
           W1l, b1l, W1r, W2l, b2l, W2r,
           Wc1, bc1, Wc2, bc2, Wc3, bc3):
    src = message_edge_index[0]
    dst = message_edge_index[1]
    qsrc = query_edge_index[0]
    qdst = query_edge_index[1]
    zrows = jnp.zeros((NP, H), jnp.float32)
    zdeg = jnp.zeros((NP, 16), jnp.float32)
    onesrow = jnp.zeros((KSEG, 16), jnp.float32).at[:, 0].set(1.0)

    agg1p, degp = _sc_seg_sum_deg(x, src, dst, zrows, zdeg, onesrow)
    h1 = _tc1(agg1p, degp, x, W1l, b1l[None, :], W1r)

    agg2p, = _sc_seg_sum(h1, src, dst, zrows, zdeg, onesrow)
    Wsd = jnp.concatenate([Wc1[:H], Wc1[H:2 * H]], axis=1)
    a_tab, b_tab = _tc2(agg2p, degp, h1, W2l, b2l[None, :], W2r, Wsd)

    q = _sc_query_gather(a_tab, b_tab, qsrc, qdst)
    q2 = q.reshape(EH, 128)
    Wa = Wc1[2 * H:]
    w2a = jnp.zeros((2 * DE, 128), jnp.float32)
    w2a = w2a.at[:DE, :64].set(Wa).at[DE:, 64:].set(Wa)
    w2c2 = jnp.zeros((128, 64), jnp.float32)
    w2c2 = w2c2.at[:64, :32].set(Wc2).at[64:, 32:].set(Wc2)
    bc1_2 = jnp.concatenate([bc1, bc1])[None, :]
    bc2_2 = jnp.concatenate([bc2, bc2])[None, :]
    oute, outo = _tc_mlp(q2, query_edge_attr.reshape(EH, 2 * DE),
                         w2a, bc1_2, w2c2, bc2_2,
                         Wc3.reshape(1, 32), bc3.reshape(1, 1))
    out = jnp.stack([oute.reshape(EH), outo.reshape(EH)], axis=-1)
    return out.reshape(EQ)


# speedup vs baseline: 1.1288x; 1.0002x over previous
import functools

import jax
import jax.numpy as jnp
from jax import lax
from jax.experimental import pallas as pl
from jax.experimental.pallas import tpu as pltpu
from jax.experimental.pallas import tpu_sc as plsc

N = 10000
D = 128
H = 128
E = 320000
EQ = 320000
DE = 16

NP = 10000
NC = 2
NS = 16
NW = NC * NS
K = 80
ROWS_PER_S = NP // NS
NCHUNK = E // NW // K

_mesh = plsc.VectorSubcoreMesh(core_axis_name="c", subcore_axis_name="s")
_sc_params = pltpu.CompilerParams(use_tc_tiling_on_sc=False)


KSEG = 128
TK = (E // NW) % KSEG


def _make_seg_sum(want_deg):
    out_type = [jax.ShapeDtypeStruct((NC, NP, H), jnp.float32)]
    scratch = [
        pltpu.VMEM((KSEG,), jnp.int32), pltpu.VMEM((KSEG,), jnp.int32),
        pltpu.VMEM((KSEG,), jnp.int32), pltpu.VMEM((KSEG,), jnp.int32),
        pltpu.VMEM((KSEG,), jnp.int32), pltpu.VMEM((KSEG,), jnp.int32),
        pltpu.VMEM((KSEG, H), jnp.float32), pltpu.VMEM((KSEG, H), jnp.float32),
        pltpu.VMEM((TK,), jnp.int32), pltpu.VMEM((TK,), jnp.int32),
        pltpu.VMEM((TK, H), jnp.float32),
        pltpu.VMEM_SHARED((NP, H), jnp.float32),
        pltpu.SemaphoreType.DMA, pltpu.SemaphoreType.DMA,
        pltpu.SemaphoreType.DMA, pltpu.SemaphoreType.DMA,
        pltpu.SemaphoreType.DMA, pltpu.SemaphoreType.DMA,
        pltpu.SemaphoreType.DMA,
    ]
    if want_deg:
        out_type.append(jax.ShapeDtypeStruct((NC, NP, 16), jnp.float32))
        scratch += [
            pltpu.VMEM((KSEG, 16), jnp.float32),
            pltpu.VMEM_SHARED((NP, 16), jnp.float32),
            pltpu.SemaphoreType.DMA,
        ]

    def body(x_hbm, src_hbm, dst_hbm, zrows_hbm, zdeg_hbm, onesrow_hbm,
             *refs):
        if want_deg:
            (agg_out, deg_out,
             si0, di0, si1, di1, si2, di2, rows0, rows1,
             tsi, tdi, trows, agg_sh,
             g0, g1, s0, s1, i0, i1, i2, ones_v, deg_sh, dsem) = refs
        else:
            (agg_out,
             si0, di0, si1, di1, si2, di2, rows0, rows1,
             tsi, tdi, trows, agg_sh,
             g0, g1, s0, s1, i0, i1, i2) = refs
        sidx = [si0, si1, si2]
        didx = [di0, di1, di2]
        rows = [rows0, rows1]
        gsem = [g0, g1]
        ssem = [s0, s1]
        isem = [i0, i1, i2]
        c = lax.axis_index("c")
        s = lax.axis_index("s")
        w = s * NC + c
        base0 = w * (E // NW)
        K = KSEG
        n = E // NW // K

        sl = pl.ds(s * ROWS_PER_S, ROWS_PER_S)
        pltpu.sync_copy(zrows_hbm.at[sl], agg_sh.at[sl])
        if want_deg:
            pltpu.sync_copy(zdeg_hbm.at[sl], deg_sh.at[sl])
            pltpu.sync_copy(onesrow_hbm, ones_v)
        plsc.subcore_barrier()

        def fire_idx(i, p, base=None):
            b = base0 + i * K if base is None else base
            pltpu.async_copy(src_hbm.at[pl.ds(b, K)], sidx[p], isem[p])
            pltpu.async_copy(dst_hbm.at[pl.ds(b, K)], didx[p], isem[p])

        def wait_idx(p):
            pltpu.make_async_copy(src_hbm.at[pl.ds(base0, K)], sidx[p],
                                  isem[p]).wait()
            pltpu.make_async_copy(dst_hbm.at[pl.ds(base0, K)], didx[p],
                                  isem[p]).wait()

        def fire_g(p, b):
            pltpu.async_copy(x_hbm.at[sidx[p]], rows[b], gsem[b])

        def wait_g(p, b):
            pltpu.make_async_copy(x_hbm.at[sidx[p]], rows[b], gsem[b]).wait()

        def fire_s(p, b):
            pltpu.async_copy(rows[b], agg_sh.at[didx[p]], ssem[b], add=True)

        def wait_s(p, b):
            pltpu.make_async_copy(rows[b], agg_sh.at[didx[p]], ssem[b]).wait()

        def fire_deg(p):
            if want_deg:
                pltpu.async_copy(ones_v, deg_sh.at[didx[p]], dsem, add=True)

        def wait_deg(p):
            if want_deg:
                pltpu.make_async_copy(ones_v, deg_sh.at[didx[p]], dsem).wait()

        def step(ci, pk, fire_next_g, fire_next_idx, drain_prev):
            p, b = pk % 3, pk % 2
            wait_g(p, b)
            fire_s(p, b)
            fire_deg(p)
            if drain_prev:
                wait_s((pk - 1) % 3, (pk - 1) % 2)
                wait_deg((pk - 1) % 3)
            if fire_next_g:
                wait_idx((pk + 1) % 3)
                fire_g((pk + 1) % 3, (pk + 1) % 2)
            if fire_next_idx:
                fire_idx(ci + 2, (pk + 2) % 3)

        fire_idx(0, 0)
        fire_idx(1, 1)
        wait_idx(0)
        fire_g(0, 0)
        step(0, 0, True, True, False)
        step(1, 1, True, True, True)

        def six(j, carry):
            ci0 = 6 * j + 2
            for k in range(6):
                step(ci0 + k, 2 + k, True, True, True)
            return carry

        lax.fori_loop(0, (n - 5) // 6, six, 0)
        for ci in range(2 + 6 * ((n - 5) // 6), n):
            step(ci, ci, ci + 1 < n, ci + 2 < n, True)
        wait_s((n - 1) % 3, (n - 1) % 2)
        wait_deg((n - 1) % 3)

        if TK:
            tb = base0 + n * K
            pltpu.sync_copy(src_hbm.at[pl.ds(tb, TK)], tsi)
            pltpu.sync_copy(dst_hbm.at[pl.ds(tb, TK)], tdi)
            pltpu.async_copy(x_hbm.at[tsi], trows, g0).wait()
            cp = pltpu.async_copy(trows, agg_sh.at[tdi], s0, add=True)
            if want_deg:
                pltpu.async_copy(ones_v.at[pl.ds(0, TK)], deg_sh.at[tdi],
                                 dsem, add=True).wait()
            cp.wait()

        plsc.subcore_barrier()
        pltpu.sync_copy(agg_sh.at[sl], agg_out.at[c, sl])
        if want_deg:
            pltpu.sync_copy(deg_sh.at[sl], deg_out.at[c, sl])

    return pl.kernel(body, out_type=out_type, mesh=_mesh,
                     scratch_types=scratch, compiler_params=_sc_params)


_sc_seg_sum_deg = _make_seg_sum(True)
_sc_seg_sum = _make_seg_sum(False)


@functools.partial(
    pl.kernel,
    out_type=jax.ShapeDtypeStruct((EQ, 64), jnp.float32),
    mesh=_mesh,
    scratch_types=[
        pltpu.VMEM((KSEG,), jnp.int32), pltpu.VMEM((KSEG,), jnp.int32),
        pltpu.VMEM((KSEG,), jnp.int32), pltpu.VMEM((KSEG,), jnp.int32),
        pltpu.VMEM((KSEG,), jnp.int32), pltpu.VMEM((KSEG,), jnp.int32),
        pltpu.VMEM((KSEG, 64), jnp.float32), pltpu.VMEM((KSEG, 64), jnp.float32),
        pltpu.VMEM((KSEG, 64), jnp.float32), pltpu.VMEM((KSEG, 64), jnp.float32),
        pltpu.VMEM((KSEG, 64), jnp.float32), pltpu.VMEM((KSEG, 64), jnp.float32),
        pltpu.VMEM((TK,), jnp.int32), pltpu.VMEM((TK,), jnp.int32),
        pltpu.VMEM((TK, 64), jnp.float32), pltpu.VMEM((TK, 64), jnp.float32),
        pltpu.SemaphoreType.DMA, pltpu.SemaphoreType.DMA,
        pltpu.SemaphoreType.DMA, pltpu.SemaphoreType.DMA,
        pltpu.SemaphoreType.DMA, pltpu.SemaphoreType.DMA,
        pltpu.SemaphoreType.DMA, pltpu.SemaphoreType.DMA,
        pltpu.SemaphoreType.DMA,
    ],
    compiler_params=_sc_params,
)
def _sc_query_gather(a_hbm, b_hbm, qsrc_hbm, qdst_hbm,
                     q_out,
                     si0, di0, si1, di1, si2, di2, a0, b0, a1, b1,
                     ov0, ov1, tsi, tdi, ta_v, tb_v,
                     ga0, gb0, ga1, gb1, o0, o1, is0, is1, is2):
    sidx = [si0, si1, si2]
    didx = [di0, di1, di2]
    av = [a0, a1]
    bv = [b0, b1]
    ov = [ov0, ov1]
    gas = [ga0, ga1]
    gbs = [gb0, gb1]
    osem = [o0, o1]
    isem = [is0, is1, is2]
    c = lax.axis_index("c")
    s = lax.axis_index("s")
    w = s * NC + c
    base0 = w * (EQ // NW)
    K = KSEG
    n = EQ // NW // K

    def fire_idx(i, p):
        b = base0 + i * K
        pltpu.async_copy(qsrc_hbm.at[pl.ds(b, K)], sidx[p], isem[p])
        pltpu.async_copy(qdst_hbm.at[pl.ds(b, K)], didx[p], isem[p])

    def wait_idx(p):
        pltpu.make_async_copy(qsrc_hbm.at[pl.ds(base0, K)], sidx[p],
                              isem[p]).wait()
        pltpu.make_async_copy(qdst_hbm.at[pl.ds(base0, K)], didx[p],
                              isem[p]).wait()

    def fire_g(p, b):
        pltpu.async_copy(a_hbm.at[sidx[p]], av[b], gas[b])
        pltpu.async_copy(b_hbm.at[didx[p]], bv[b], gbs[b])

    def wait_g(p, b):
        pltpu.make_async_copy(a_hbm.at[sidx[p]], av[b], gas[b]).wait()
        pltpu.make_async_copy(b_hbm.at[didx[p]], bv[b], gbs[b]).wait()

    def add_rows(b):
        a_v, b_v, o_v = av[b], bv[b], ov[b]

        def row4(r4, carry):
            r0 = r4 * 4
            for k in range(4):
                for l in range(4):
                    slc = pl.ds(l * 16, 16)
                    o_v[r0 + k, slc] = a_v[r0 + k, slc] + b_v[r0 + k, slc]
            return carry
        lax.fori_loop(0, K // 4, row4, 0)

    def fire_out(ci, b):
        pltpu.async_copy(ov[b], q_out.at[pl.ds(base0 + ci * K, K)], osem[b])

    def wait_out(b):
        pltpu.make_async_copy(ov[b], q_out.at[pl.ds(base0, K)],
                              osem[b]).wait()

    def step(ci, pk, fire_next_g, fire_next_idx, drain_prev):
        p, b = pk % 3, pk % 2
        wait_g(p, b)
        if fire_next_g:
            wait_idx((pk + 1) % 3)
            fire_g((pk + 1) % 3, (pk + 1) % 2)
        if fire_next_idx:
            fire_idx(ci + 2, (pk + 2) % 3)
        if drain_prev:
            wait_out((pk - 1) % 2)
        add_rows(b)
        fire_out(ci, b)

    fire_idx(0, 0)
    fire_idx(1, 1)
    wait_idx(0)
    fire_g(0, 0)
    step(0, 0, True, True, False)
    step(1, 1, True, True, True)

    def six(j, carry):
        ci0 = 6 * j + 2
        for k in range(6):
            step(ci0 + k, 2 + k, True, True, True)
        return carry

    lax.fori_loop(0, (n - 5) // 6, six, 0)
    for ci in range(2 + 6 * ((n - 5) // 6), n):
        step(ci, ci, ci + 1 < n, ci + 2 < n, True)
    wait_out((n - 1) % 2)

    if TK:
        tb = base0 + n * K
        pltpu.sync_copy(qsrc_hbm.at[pl.ds(tb, TK)], tsi)
        pltpu.sync_copy(qdst_hbm.at[pl.ds(tb, TK)], tdi)
        cpa = pltpu.async_copy(a_hbm.at[tsi], ta_v, ga0)
        cpb = pltpu.async_copy(b_hbm.at[tdi], tb_v, gb0)
        cpa.wait()
        cpb.wait()
        for r in range(TK):
            for l in range(4):
                slc = pl.ds(l * 16, 16)
                ta_v[r, slc] = ta_v[r, slc] + tb_v[r, slc]
        pltpu.sync_copy(ta_v, q_out.at[pl.ds(tb, TK)])


BN = 1000


def _tc1_body(aggp_ref, degp_ref, x_ref, wl_ref, bl_ref, wr_ref, h1_ref):
    agg = aggp_ref[0] + aggp_ref[1]
    deg = degp_ref[0, :, 0] + degp_ref[1, :, 0]
    rdeg = 1.0 / jnp.maximum(deg, 1.0)
    mean = agg * rdeg[:, None]
    out = (jnp.dot(mean, wl_ref[...], preferred_element_type=jnp.float32)
           + bl_ref[...]
           + jnp.dot(x_ref[...], wr_ref[...], preferred_element_type=jnp.float32))
    h1_ref[...] = jnp.maximum(out, 0.0)


def _tc1(aggp, degp, x, wl, bl, wr):
    return pl.pallas_call(
        _tc1_body,
        grid=(NP // BN,),
        in_specs=[
            pl.BlockSpec((NC, BN, H), lambda i: (0, i, 0)),
            pl.BlockSpec((NC, BN, 16), lambda i: (0, i, 0)),
            pl.BlockSpec((BN, H), lambda i: (i, 0)),
            pl.BlockSpec((H, H), lambda i: (0, 0)),
            pl.BlockSpec((1, H), lambda i: (0, 0)),
            pl.BlockSpec((H, H), lambda i: (0, 0)),
        ],
        out_specs=pl.BlockSpec((BN, H), lambda i: (i, 0)),
        out_shape=jax.ShapeDtypeStruct((NP, H), jnp.float32),
    )(aggp, degp, x, wl, bl, wr)


def _tc2_body(aggp_ref, degp_ref, h1_ref, wl_ref, bl_ref, wr_ref, wsd_ref,
              a_ref, b_ref):
    agg = aggp_ref[0] + aggp_ref[1]
    deg = degp_ref[0, :, 0] + degp_ref[1, :, 0]
    rdeg = 1.0 / jnp.maximum(deg, 1.0)
    mean = agg * rdeg[:, None]
    h2 = (jnp.dot(mean, wl_ref[...], preferred_element_type=jnp.float32)
          + bl_ref[...]
          + jnp.dot(h1_ref[...], wr_ref[...], preferred_element_type=jnp.float32))
    ab = jnp.dot(h2, wsd_ref[...], preferred_element_type=jnp.float32)
    a_ref[...] = ab[:, :64]
    b_ref[...] = ab[:, 64:]


def _tc2(aggp, degp, h1, wl, bl, wr, wsd):
    return pl.pallas_call(
        _tc2_body,
        grid=(NP // BN,),
        in_specs=[
            pl.BlockSpec((NC, BN, H), lambda i: (0, i, 0)),
            pl.BlockSpec((NC, BN, 16), lambda i: (0, i, 0)),
            pl.BlockSpec((BN, H), lambda i: (i, 0)),
            pl.BlockSpec((H, H), lambda i: (0, 0)),
            pl.BlockSpec((1, H), lambda i: (0, 0)),
            pl.BlockSpec((H, H), lambda i: (0, 0)),
            pl.BlockSpec((H, H), lambda i: (0, 0)),
        ],
        out_specs=[
            pl.BlockSpec((BN, 64), lambda i: (i, 0)),
            pl.BlockSpec((BN, 64), lambda i: (i, 0)),
        ],
        out_shape=[
            jax.ShapeDtypeStruct((NP, 64), jnp.float32),
            jax.ShapeDtypeStruct((NP, 64), jnp.float32),
        ],
    )(aggp, degp, h1, wl, bl, wr, wsd)


EH = EQ // 2
BQ = 16000
GQ = EH // BQ
BE = 2000
BR = 8


def _tc_mlp_body(q_ref, attr_ref, w2a_ref, bc1_ref, w2c2_ref, bc2_ref,
                 wc3_ref, bc3_ref, oute_ref, outo_ref):
    q = q_ref[...]
    attr = attr_ref[...]
    z1 = q + jnp.dot(attr, w2a_ref[...], preferred_element_type=jnp.float32) + bc1_ref[...]
    z1 = jnp.maximum(z1, 0.0)
    z2 = jnp.dot(z1, w2c2_ref[...], preferred_element_type=jnp.float32) + bc2_ref[...]
    z2 = jnp.maximum(z2, 0.0)
    ze = jnp.sum(z2[:, :32] * wc3_ref[...], axis=1) + bc3_ref[0, 0]
    zo = jnp.sum(z2[:, 32:] * wc3_ref[...], axis=1) + bc3_ref[0, 0]
    oute_ref[...] = ze.reshape(BR, BE)
    outo_ref[...] = zo.reshape(BR, BE)


def _tc_mlp(q2, attr2, w2a, bc1_2, w2c2, bc2_2, wc3, bc3):
    return pl.pallas_call(
        _tc_mlp_body,
        grid=(GQ,),
        in_specs=[
            pl.BlockSpec((BQ, 128), lambda i: (i, 0)),
            pl.BlockSpec((BQ, 2 * DE), lambda i: (i, 0)),
            pl.BlockSpec((2 * DE, 128), lambda i: (0, 0)),
            pl.BlockSpec((1, 128), lambda i: (0, 0)),
            pl.BlockSpec((128, 64), lambda i: (0, 0)),
            pl.BlockSpec((1, 64), lambda i: (0, 0)),
            pl.BlockSpec((1, 32), lambda i: (0, 0)),
            pl.BlockSpec((1, 1), lambda i: (0, 0)),
        ],
        out_specs=[
            pl.BlockSpec((BR, BE), lambda i: (i, 0)),
            pl.BlockSpec((BR, BE), lambda i: (i, 0)),
        ],
        out_shape=[
            jax.ShapeDtypeStruct((GQ * BR, BE), jnp.float32),
            jax.ShapeDtypeStruct((GQ * BR, BE), jnp.float32),
        ],
    )(q2, attr2, w2a, bc1_2, w2c2, bc2_2, wc3, bc3)


def kernel(x, message_edge_index, query_edge_index, query_edge_attr,
           W1l, b1l, W1r, W2l, b2l, W2r,
           Wc1, bc1, Wc2, bc2, Wc3, bc3):
    src = message_edge_index[0]
    dst = message_edge_index[1]
    qsrc = query_edge_index[0]
    qdst = query_edge_index[1]
    zrows = jnp.zeros((NP, H), jnp.float32)
    zdeg = jnp.zeros((NP, 16), jnp.float32)
    onesrow = jnp.zeros((KSEG, 16), jnp.float32).at[:, 0].set(1.0)

    agg1p, degp = _sc_seg_sum_deg(x, src, dst, zrows, zdeg, onesrow)
    h1 = _tc1(agg1p, degp, x, W1l, b1l[None, :], W1r)

    agg2p, = _sc_seg_sum(h1, src, dst, zrows, zdeg, onesrow)
    Wsd = jnp.concatenate([Wc1[:H], Wc1[H:2 * H]], axis=1)
    a_tab, b_tab = _tc2(agg2p, degp, h1, W2l, b2l[None, :], W2r, Wsd)

    q = _sc_query_gather(a_tab, b_tab, qsrc, qdst)
    q2 = q.reshape(EH, 128)
    Wa = Wc1[2 * H:]
    w2a = jnp.zeros((2 * DE, 128), jnp.float32)
    w2a = w2a.at[:DE, :64].set(Wa).at[DE:, 64:].set(Wa)
    w2c2 = jnp.zeros((128, 64), jnp.float32)
    w2c2 = w2c2.at[:64, :32].set(Wc2).at[64:, 32:].set(Wc2)
    bc1_2 = jnp.concatenate([bc1, bc1])[None, :]
    bc2_2 = jnp.concatenate([bc2, bc2])[None, :]
    oute, outo = _tc_mlp(q2, query_edge_attr.reshape(EH, 2 * DE),
                         w2a, bc1_2, w2c2, bc2_2,
                         Wc3.reshape(1, 32), bc3.reshape(1, 1))
    out = jnp.stack([oute.reshape(EH), outo.reshape(EH)], axis=-1)
    return out.reshape(EQ)
